# baseline (speedup 1.0000x reference)
import functools
import jax
import jax.numpy as jnp
from jax import lax
from jax.experimental import pallas as pl
from jax.experimental.pallas import tpu as pltpu
from jax.experimental.pallas import tpu_sc as plsc

SH = 128
VHC = 16
W = SH + 3 * VHC
SEW = 32
EBLK = 1280
NBLK = 1000
EBLK2 = 2000
GCH = 80
SCH = 128
EPAD = 20480


def _bd3(w):
    return jnp.kron(jnp.eye(3, dtype=w.dtype), w)


def _summ(h):
    return jnp.kron(jnp.ones((3, 1), dtype=jnp.float32), jnp.eye(h, dtype=jnp.float32))


def _ln_s(s, w, b):
    mu = jnp.mean(s, axis=-1, keepdims=True)
    var = jnp.mean((s - mu) * (s - mu), axis=-1, keepdims=True)
    return (s - mu) / jnp.sqrt(var + 1e-5) * w + b


def _dot(x, w):
    return jnp.dot(x, w, preferred_element_type=jnp.float32)


def _vnorm(vh, summ):
    return jnp.sqrt(jnp.clip(_dot(vh * vh, summ), 1e-8, None))


def _vgate(vo, summ):
    sig = jax.nn.sigmoid(_vnorm(vo, summ))
    return vo * jnp.concatenate([sig, sig, sig], axis=1)


def _node_embed_kernel(s_ref, v_ref, lnw, lnb, whb, summ, wss, wsn, wsb, wvb,
                       out_ref):
    s = _ln_s(s_ref[...], lnw[...], lnb[...])
    v = v_ref[...]
    vn = jnp.sqrt(jnp.sum(v * v, axis=-1, keepdims=True) / 3.0 + 1e-8)
    v = v / vn
    vh = _dot(v, whb[...])
    nrm = _vnorm(vh, summ[...])
    so = _dot(s, wss[...]) + _dot(nrm, wsn[...]) + wsb[...]
    vo = _dot(vh, wvb[...])
    out_ref[...] = jnp.concatenate([so, vo], axis=1)


def _edge_embed_kernel(s_ref, v_ref, lnw, lnb, wh00, wss, wsn, wsb, wv00,
                       so_ref, vo_ref):
    s = _ln_s(s_ref[...], lnw[...], lnb[...])
    v = v_ref[...]
    vn = jnp.sqrt(jnp.sum(v * v, axis=-1, keepdims=True) + 1e-8)
    v = v / vn
    vh = v * wh00[0, 0]
    nrm = jnp.sqrt(jnp.clip(jnp.sum(vh * vh, axis=-1, keepdims=True), 1e-8, None))
    so_ref[...] = _dot(s, wss[...]) + _dot(nrm, wsn[...]) + wsb[...]
    vo_ref[...] = vh * wv00[0, 0]


def _node_pre_kernel(x_ref, wa, wsrc, wc, wdst, a_ref, c_ref):
    x = x_ref[...]
    s, v = x[:, :SH], x[:, SH:]
    blk = s.shape[0]
    pad = jnp.zeros((blk, 256 - SH - 99), jnp.float32)
    a_ref[...] = jnp.concatenate([_dot(s, wa[...]), _dot(v, wsrc[...]), pad], 1)
    c_ref[...] = jnp.concatenate([_dot(s, wc[...]), _dot(v, wdst[...]), pad], 1)


def _message_kernel(ga_ref, gc_ref, es_ref, ev_ref,
                    w0ev, summ33, ws0es, ws0n, ws0bias,
                    wv0b, summ16,
                    wh1b, ws1s, ws1n, ws1bias, wv1b,
                    wh2b, ws2s, ws2n, ws2bias, wv2b,
                    ms_ref, mv_ref):
    ga, gc = ga_ref[...], gc_ref[...]
    es, ev = es_ref[...], ev_ref[...]
    vh0 = ga[:, SH:SH + 99] + gc[:, SH:SH + 99] + _dot(ev, w0ev[...])
    nrm0 = _vnorm(vh0, summ33[...])
    s0 = (ga[:, :SH] + gc[:, :SH] + _dot(es, ws0es[...])
          + _dot(nrm0, ws0n[...]) + ws0bias[...])
    s0 = jnp.maximum(s0, 0.0)
    v0 = _vgate(_dot(vh0, wv0b[...]), summ16[...])
    vh1 = _dot(v0, wh1b[...])
    nrm1 = _vnorm(vh1, summ16[...])
    s1 = jnp.maximum(_dot(s0, ws1s[...]) + _dot(nrm1, ws1n[...]) + ws1bias[...], 0.0)
    v1 = _vgate(_dot(vh1, wv1b[...]), summ16[...])
    vh2 = _dot(v1, wh2b[...])
    nrm2 = _vnorm(vh2, summ16[...])
    s2 = _dot(s1, ws2s[...]) + _dot(nrm2, ws2n[...]) + ws2bias[...]
    v2 = _dot(vh2, wv2b[...])
    ms_ref[...] = s2
    mv_ref[...] = jnp.concatenate(
        [v2, jnp.zeros((v2.shape[0], 128 - 3 * VHC), jnp.float32)], axis=1)


def _cnt_kernel(c_ref, out_ref):
    c = c_ref[...]
    out_ref[...] = jnp.maximum(c[0][:, :16], 1.0)


def _node_update_kernel(x_ref, p_ref, c_ref,
                        ln0w, ln0b,
                        f0wh, summ32, f0wss, f0wsn, f0wsb, f0wv,
                        f1wh, f1wss, f1wsn, f1wsb, f1wv, summ16,
                        ln1w, ln1b,
                        out_ref):
    x = x_ref[...]
    cnt = c_ref[...][:, :1]
    p = p_ref[...]
    s = x[:, :SH] + p[0] / cnt
    v = x[:, SH:] + p[1][:, :3 * VHC] / cnt
    s = _ln_s(s, ln0w[...], ln0b[...])
    vn = jnp.sqrt(jnp.sum(v * v, axis=-1, keepdims=True) / VHC + 1e-8)
    v = v / vn
    vh = _dot(v, f0wh[...])
    nrm = _vnorm(vh, summ32[...])
    fs = jnp.maximum(_dot(s, f0wss[...]) + _dot(nrm, f0wsn[...]) + f0wsb[...], 0.0)
    fv = _vgate(_dot(vh, f0wv[...]), summ32[...])
    vh1 = _dot(fv, f1wh[...])
    nrm1 = _vnorm(vh1, summ32[...])
    fs1 = _dot(fs, f1wss[...]) + _dot(nrm1, f1wsn[...]) + f1wsb[...]
    fv1 = _dot(vh1, f1wv[...])
    s = s + fs1
    v = v + fv1
    s = _ln_s(s, ln1w[...], ln1b[...])
    vn = jnp.sqrt(jnp.sum(v * v, axis=-1, keepdims=True) / VHC + 1e-8)
    v = v / vn
    out_ref[...] = jnp.concatenate([s, v], axis=1)


def _heads_pre_kernel(x_ref, av_ref,
                      plnw, plnb, pwh, summ16, pwss, pwsn, pwsb,
                      vlnw, vlnb, vwh, vwss, vwsn, vwsb,
                      pol_ref, val_ref):
    x = x_ref[...]
    s, v = x[:, :SH], x[:, SH:]

    def head(lnw, lnb, whb, wss, wsn, wsb):
        s2 = _ln_s(s, lnw, lnb)
        vn = jnp.sqrt(jnp.sum(v * v, axis=-1, keepdims=True) / VHC + 1e-8)
        v2 = v / vn
        vh = _dot(v2, whb)
        nrm = _vnorm(vh, summ16[...])
        return _dot(s2, wss) + _dot(nrm, wsn) + wsb

    pol_ref[...] = head(plnw[...], plnb[...], pwh[...], pwss[...], pwsn[...],
                        pwsb[...]) * av_ref[...]
    val_ref[...] = head(vlnw[...], vlnb[...], vwh[...], vwss[...], vwsn[...],
                        vwsb[...])


def _matmul_bias_kernel(x_ref, w_ref, b_ref, out_ref, *, act):
    h = _dot(x_ref[...], w_ref[...]) + b_ref[...]
    if act:
        h = jnp.maximum(h, 0.0)
    out_ref[...] = h


def _val_head_kernel(vp_ref, w1, b1, w2, b2, out_ref):
    vsum = jnp.sum(vp_ref[...], axis=1)
    h = _dot(vsum, w1[...]) + b1[...]
    h = jnp.where(h > 0, h, 0.01 * h)
    out_ref[...] = _dot(h, w2[...]) + b2[...]


def _sc_gather2(tab_a, tab_c, idx_src, idx_dst):
    n, w = tab_a.shape
    e = idx_src.shape[0]
    info = plsc.get_sparse_core_info()
    nw = info.num_cores * info.num_subcores
    per_w = e // nw
    n_ch = per_w // GCH
    mesh = plsc.VectorSubcoreMesh(core_axis_name="c", subcore_axis_name="s")

    @functools.partial(
        pl.kernel, mesh=mesh,
        out_type=[jax.ShapeDtypeStruct((e, w), jnp.float32),
                  jax.ShapeDtypeStruct((e, w), jnp.float32)],
        scratch_types=[pltpu.VMEM((per_w,), jnp.int32),
                       pltpu.VMEM((per_w,), jnp.int32),
                       pltpu.VMEM((GCH, w), jnp.float32),
                       pltpu.VMEM((GCH, w), jnp.float32),
                       pltpu.VMEM((GCH, w), jnp.float32),
                       pltpu.VMEM((GCH, w), jnp.float32),
                       pltpu.SemaphoreType.DMA, pltpu.SemaphoreType.DMA,
                       pltpu.SemaphoreType.DMA, pltpu.SemaphoreType.DMA,
                       pltpu.SemaphoreType.DMA, pltpu.SemaphoreType.DMA,
                       pltpu.SemaphoreType.DMA, pltpu.SemaphoreType.DMA],
    )
    def k(ta_h, tc_h, src_h, dst_h, oa_h, oc_h,
          isv, idv, ra0, ra1, rc0, rc1,
          sga0, sga1, sgc0, sgc1, swa0, swa1, swc0, swc1):
        wid = lax.axis_index("c") * info.num_subcores + lax.axis_index("s")
        base = wid * per_w
        ra, rc = (ra0, ra1), (rc0, rc1)
        sga, sgc = (sga0, sga1), (sgc0, sgc1)
        swa, swc = (swa0, swa1), (swc0, swc1)
        pltpu.sync_copy(src_h.at[pl.ds(base, per_w)], isv)
        pltpu.sync_copy(dst_h.at[pl.ds(base, per_w)], idv)

        def start_gather(i, p):
            off = i * GCH
            pltpu.async_copy(ta_h.at[isv.at[pl.ds(off, GCH)]], ra[p], sga[p])
            pltpu.async_copy(tc_h.at[idv.at[pl.ds(off, GCH)]], rc[p], sgc[p])

        def wait_writeouts(p):
            pltpu.make_async_copy(ra[p], oa_h.at[pl.ds(base, GCH)], swa[p]).wait()
            pltpu.make_async_copy(rc[p], oc_h.at[pl.ds(base, GCH)], swc[p]).wait()

        def body(i2, carry):
            i0 = i2 * 2
            for p in (0, 1):
                @pl.when(i2 >= 1)
                def _():
                    wait_writeouts(p)
                start_gather(i0 + p, p)
            for p in (0, 1):
                pltpu.make_async_copy(
                    ta_h.at[isv.at[pl.ds(0, GCH)]], ra[p], sga[p]).wait()
                pltpu.make_async_copy(
                    tc_h.at[idv.at[pl.ds(0, GCH)]], rc[p], sgc[p]).wait()
                off = base + (i0 + p) * GCH
                pltpu.async_copy(ra[p], oa_h.at[pl.ds(off, GCH)], swa[p])
                pltpu.async_copy(rc[p], oc_h.at[pl.ds(off, GCH)], swc[p])
            return carry

        lax.fori_loop(0, n_ch // 2, body, 0)
        for p in (0, 1):
            wait_writeouts(p)

    return k(tab_a, tab_c, idx_src, idx_dst)


def _sc_scatter_cols(ms, mv, idx, zeros_blk):
    e = ms.shape[0]
    info = plsc.get_sparse_core_info()
    ns = info.num_subcores
    per_t = e // ns
    n_ch = per_t // SCH
    rpt = zeros_blk.shape[0]
    npad = rpt * ns
    idx2d = idx.reshape(e // SCH, SCH)
    mesh = plsc.VectorSubcoreMesh(core_axis_name="c", subcore_axis_name="s")

    @functools.partial(
        pl.kernel, mesh=mesh,
        out_type=jax.ShapeDtypeStruct((2, npad, 128), jnp.float32),
        scratch_types=[pltpu.VMEM((n_ch, SCH), jnp.int32),
                       pltpu.VMEM((SCH, 128), jnp.float32),
                       pltpu.VMEM((SCH, 128), jnp.float32),
                       pltpu.VMEM_SHARED((npad, 128), jnp.float32),
                       pltpu.SemaphoreType.DMA, pltpu.SemaphoreType.DMA,
                       pltpu.SemaphoreType.DMA, pltpu.SemaphoreType.DMA],
    )
    def k(ms_h, mv_h, idx_h, zeros_h, out_h, idx_v, r0, r1, acc,
          src0, src1, ssa0, ssa1):
        c = lax.axis_index("c")
        s = lax.axis_index("s")
        rows = (r0, r1)
        src_sem = (src0, src1)
        sa_sem = (ssa0, ssa1)
        pltpu.sync_copy(zeros_h, acc.at[pl.ds(s * rpt, rpt)])
        pltpu.sync_copy(idx_h.at[pl.ds(s * n_ch, n_ch)], idx_v)
        plsc.subcore_barrier()

        def start_rowcopy(j, p):
            b = s * per_t + j * SCH

            @pl.when(c == 0)
            def _():
                pltpu.async_copy(ms_h.at[pl.ds(b, SCH)], rows[p], src_sem[p])

            @pl.when(c == 1)
            def _():
                pltpu.async_copy(mv_h.at[pl.ds(b, SCH)], rows[p], src_sem[p])

        def body(j2, carry):
            j0 = j2 * 2
            for p in (0, 1):
                @pl.when(j2 >= 1)
                def _():
                    pltpu.make_async_copy(
                        rows[p], acc.at[idx_v.at[0]], sa_sem[p]).wait()
                start_rowcopy(j0 + p, p)
            for p in (0, 1):
                pltpu.make_async_copy(
                    ms_h.at[pl.ds(0, SCH)], rows[p], src_sem[p]).wait()
                pltpu.async_copy(rows[p], acc.at[idx_v.at[j0 + p]],
                                 sa_sem[p], add=True)
            return carry

        lax.fori_loop(0, n_ch // 2, body, 0)
        for p in (0, 1):
            pltpu.make_async_copy(rows[p], acc.at[idx_v.at[0]], sa_sem[p]).wait()
        plsc.subcore_barrier()
        pltpu.sync_copy(acc.at[pl.ds(s * rpt, rpt)],
                        out_h.at[c, pl.ds(s * rpt, rpt)])

    return k(ms, mv, idx2d, zeros_blk)


def _sc_counts(idx, e, ones_blk, zeros_blk):
    info = plsc.get_sparse_core_info()
    ns = info.num_subcores
    n_ch = e // (ns * SCH)
    rpt = zeros_blk.shape[0]
    npad = rpt * ns
    mesh = plsc.VectorSubcoreMesh(core_axis_name="c", subcore_axis_name="s")

    @functools.partial(
        pl.kernel, mesh=mesh,
        out_type=jax.ShapeDtypeStruct((2, npad, 128), jnp.float32),
        scratch_types=[pltpu.VMEM((SCH,), jnp.int32),
                       pltpu.VMEM((SCH, 128), jnp.float32),
                       pltpu.VMEM_SHARED((npad, 128), jnp.float32),
                       pltpu.SemaphoreType.DMA],
    )
    def k(idx_h, ones_h, zeros_h, out_h, idx_v, rows_v, acc, sem):
        c = lax.axis_index("c")
        s = lax.axis_index("s")
        pltpu.sync_copy(zeros_h, acc.at[pl.ds(s * rpt, rpt)])
        pltpu.sync_copy(ones_h, rows_v)
        plsc.subcore_barrier()

        def body(i, carry):
            b = (i * ns + s) * SCH
            pltpu.sync_copy(idx_h.at[pl.ds(b, SCH)], idx_v)
            pltpu.sync_copy(rows_v, acc.at[idx_v], add=True)
            return carry

        lax.fori_loop(0, n_ch, body, 0)
        plsc.subcore_barrier()
        pltpu.sync_copy(acc.at[pl.ds(s * rpt, rpt)],
                        out_h.at[c, pl.ds(s * rpt, rpt)])

    return k(idx, ones_blk, zeros_blk)


def _dbg_gather(ta, tc, i_s, i_d):
    return ta[i_s], tc[i_d]


def _dbg_scatter(ms, mv, idx, z):
    npad = z.shape[0] * 16
    s0 = jax.ops.segment_sum(ms, idx, num_segments=npad)
    s1 = jax.ops.segment_sum(mv, idx, num_segments=npad)
    return jnp.stack([s0, s1])


def _dbg_counts(idx, e, ones, z):
    npad = z.shape[0] * 16
    c = jax.ops.segment_sum(jnp.ones((e,), jnp.float32), idx, num_segments=npad)
    c = jnp.broadcast_to(c[:, None], (npad, 128))
    return jnp.stack([c, c])


def _tc_call(body, grid, in_specs, out_specs, out_shape, *args):
    return pl.pallas_call(
        body,
        grid=grid,
        in_specs=in_specs,
        out_specs=out_specs,
        out_shape=out_shape,
    )(*args)


def _full(a):
    return pl.BlockSpec(a.shape, lambda i: tuple(0 for _ in a.shape))


def kernel(node_s, node_v, edge_s, edge_v, avaliable_pos, params, edge_index,
           batch_ids, ptr):
    f32 = jnp.float32
    n = node_s.shape[0]
    e = edge_s.shape[0]
    b = ptr.shape[0] - 1
    l = n // b
    src = edge_index[0].astype(jnp.int32)
    dst = edge_index[1].astype(jnp.int32)
    ep = ((e + EPAD - 1) // EPAD) * EPAD
    rpt = (n // 16 // 8 + 1) * 8
    npad = rpt * 16
    pad = ep - e
    src_g = jnp.concatenate([src, jnp.zeros((pad,), jnp.int32)])
    dst_g = jnp.concatenate([dst, jnp.zeros((pad,), jnp.int32)])
    dst_s = jnp.concatenate([dst, jnp.full((pad,), npad - 1, jnp.int32)])

    P = params
    su16, su32, su33 = _summ(16), _summ(32), _summ(33)

    ng = P["node_gvp"]
    ne_args = (P["node_ln"]["w"][None, :], P["node_ln"]["b"][None, :],
               _bd3(ng["wh"].T), su16,
               ng["ws"]["w"].T[:SH], ng["ws"]["w"].T[SH:],
               ng["ws"]["b"][None, :], _bd3(ng["wv"].T))
    eg = P["edge_gvp"]
    ee_args = (P["edge_ln"]["w"][None, :], P["edge_ln"]["b"][None, :],
               eg["wh"], eg["ws"]["w"].T[:SEW], eg["ws"]["w"].T[SEW:],
               eg["ws"]["b"][None, :], eg["wv"])

    def pre_args(cp):
        m0 = cp["m0"]
        wh0t = m0["wh"].T
        ws0t = m0["ws"]["w"].T
        return (ws0t[0:SH], _bd3(wh0t[0:16]),
                ws0t[SH + SEW:2 * SH + SEW], _bd3(wh0t[17:33]))

    def msg_args(cp):
        m0, m1, m2 = cp["m0"], cp["m1"], cp["m2"]
        wh0t = m0["wh"].T
        ws0t = m0["ws"]["w"].T
        return (_bd3(wh0t[16:17]), su33,
                ws0t[SH:SH + SEW],
                ws0t[2 * SH + SEW:], m0["ws"]["b"][None, :],
                _bd3(m0["wv"].T), su16,
                _bd3(m1["wh"].T), m1["ws"]["w"].T[:SH], m1["ws"]["w"].T[SH:],
                m1["ws"]["b"][None, :], _bd3(m1["wv"].T),
                _bd3(m2["wh"].T), m2["ws"]["w"].T[:SH], m2["ws"]["w"].T[SH:],
                m2["ws"]["b"][None, :], _bd3(m2["wv"].T))

    def upd_args(lp):
        f0, f1 = lp["ff0"], lp["ff1"]
        return (lp["ln0"]["w"][None, :], lp["ln0"]["b"][None, :],
                _bd3(f0["wh"].T), su32,
                f0["ws"]["w"].T[:SH], f0["ws"]["w"].T[SH:],
                f0["ws"]["b"][None, :], _bd3(f0["wv"].T),
                _bd3(f1["wh"].T), f1["ws"]["w"].T[:4 * SH],
                f1["ws"]["w"].T[4 * SH:], f1["ws"]["b"][None, :],
                _bd3(f1["wv"].T), su16,
                lp["ln1"]["w"][None, :], lp["ln1"]["b"][None, :])

    pg, vg = P["pol_gvp"], P["val_gvp"]
    hp_args = (P["pol_ln"]["w"][None, :], P["pol_ln"]["b"][None, :],
               _bd3(pg["wh"].T), su16,
               pg["ws"]["w"].T[:SH], pg["ws"]["w"].T[SH:],
               pg["ws"]["b"][None, :],
               P["val_ln"]["w"][None, :], P["val_ln"]["b"][None, :],
               _bd3(vg["wh"].T),
               vg["ws"]["w"].T[:SH], vg["ws"]["w"].T[SH:],
               vg["ws"]["b"][None, :])

    nv_flat = node_v.swapaxes(1, 2).reshape(n, 9)
    ngrid = n // NBLK
    nspec = pl.BlockSpec((NBLK, W), lambda i: (i, 0))
    x = _tc_call(_node_embed_kernel, (ngrid,),
                 [pl.BlockSpec((NBLK, SH), lambda i: (i, 0)),
                  pl.BlockSpec((NBLK, 9), lambda i: (i, 0))]
                 + [_full(a) for a in ne_args],
                 nspec, jax.ShapeDtypeStruct((n, W), f32),
                 node_s, nv_flat, *ne_args)

    ev_flat = edge_v.swapaxes(1, 2).reshape(e, 3)
    egrid_e = e // EBLK
    egrid = ep // EBLK
    emax = egrid_e - 1
    es2, ev2 = _tc_call(
        _edge_embed_kernel, (egrid_e,),
        [pl.BlockSpec((EBLK, SEW), lambda i: (i, 0)),
         pl.BlockSpec((EBLK, 3), lambda i: (i, 0))]
        + [_full(a) for a in ee_args],
        [pl.BlockSpec((EBLK, SEW), lambda i: (i, 0)),
         pl.BlockSpec((EBLK, 3), lambda i: (i, 0))],
        [jax.ShapeDtypeStruct((e, SEW), f32), jax.ShapeDtypeStruct((e, 3), f32)],
        edge_s, ev_flat, *ee_args)

    ones_ch = jnp.ones((SCH, 128), f32)
    zeros128 = jnp.zeros((rpt, 128), f32)
    cnt_parts = _sc_counts(dst_s, ep, ones_ch, zeros128)
    cnt16 = _tc_call(
        _cnt_kernel, (ngrid,),
        [pl.BlockSpec((2, NBLK, 128), lambda i: (0, i, 0))],
        pl.BlockSpec((NBLK, 16), lambda i: (i, 0)),
        jax.ShapeDtypeStruct((n, 16), f32),
        cnt_parts)
    espec = pl.BlockSpec((EBLK, W), lambda i: (i, 0))

    p256 = pl.BlockSpec((NBLK, 256), lambda i: (i, 0))
    g256 = pl.BlockSpec((EBLK, 256), lambda i: (i, 0))
    for li in range(3):
        lp = P["layer%d" % li]
        pargs = pre_args(lp["conv"])
        a_t, c_t = _tc_call(
            _node_pre_kernel, (ngrid,),
            [nspec] + [_full(w) for w in pargs],
            [p256, p256],
            [jax.ShapeDtypeStruct((n, 256), f32),
             jax.ShapeDtypeStruct((n, 256), f32)],
            x, *pargs)
        ga, gc = _sc_gather2(a_t, c_t, src_g, dst_g)
        margs = msg_args(lp["conv"])
        ms, mv = _tc_call(
            _message_kernel, (egrid,),
            [g256, g256,
             pl.BlockSpec((EBLK, SEW), lambda i: (jnp.minimum(i, emax), 0)),
             pl.BlockSpec((EBLK, 3), lambda i: (jnp.minimum(i, emax), 0))]
            + [_full(a) for a in margs],
            [pl.BlockSpec((EBLK, SH), lambda i: (i, 0)),
             pl.BlockSpec((EBLK, 128), lambda i: (i, 0))],
            [jax.ShapeDtypeStruct((ep, SH), f32),
             jax.ShapeDtypeStruct((ep, 128), f32)],
            ga, gc, es2, ev2, *margs)
        parts = _sc_scatter_cols(ms, mv, dst_s, zeros128)
        uargs = upd_args(lp)
        x = _tc_call(
            _node_update_kernel, (ngrid,),
            [nspec,
             pl.BlockSpec((2, NBLK, 128), lambda i: (0, i, 0)),
             pl.BlockSpec((NBLK, 16), lambda i: (i, 0))]
            + [_full(a) for a in uargs],
            nspec, jax.ShapeDtypeStruct((n, W), f32),
            x, parts, cnt16, *uargs)

    av = avaliable_pos.reshape(n, 1)
    pol_pre, val_pre = _tc_call(
        _heads_pre_kernel, (ngrid,),
        [nspec, pl.BlockSpec((NBLK, 1), lambda i: (i, 0))]
        + [_full(a) for a in hp_args],
        [pl.BlockSpec((NBLK, 32), lambda i: (i, 0)),
         pl.BlockSpec((NBLK, 32), lambda i: (i, 0))],
        [jax.ShapeDtypeStruct((n, 32), f32), jax.ShapeDtypeStruct((n, 32), f32)],
        x, av, *hp_args)

    d_in = l * 32
    d_h = pl.cdiv(20 * l, 512) * 512
    xp = jnp.zeros((64, d_in), f32).at[:b].set(pol_pre.reshape(b, d_in))
    w1 = P["pol_fc1"]["w"].T
    w1 = jnp.zeros((d_in, d_h), f32).at[:, :20 * l].set(w1)
    b1 = jnp.zeros((1, d_h), f32).at[0, :20 * l].set(P["pol_fc1"]["b"])
    w2 = jnp.zeros((d_h, d_h), f32).at[:20 * l, :20 * l].set(P["pol_fc2"]["w"].T)
    b2 = jnp.zeros((1, d_h), f32).at[0, :20 * l].set(P["pol_fc2"]["b"])

    hgrid = d_h // 512
    h1 = _tc_call(
        functools.partial(_matmul_bias_kernel, act=True), (hgrid,),
        [pl.BlockSpec((64, d_in), lambda j: (0, 0)),
         pl.BlockSpec((d_in, 512), lambda j: (0, j)),
         pl.BlockSpec((1, 512), lambda j: (0, j))],
        pl.BlockSpec((64, 512), lambda j: (0, j)),
        jax.ShapeDtypeStruct((64, d_h), f32),
        xp, w1, b1)
    pol = _tc_call(
        functools.partial(_matmul_bias_kernel, act=False), (hgrid,),
        [pl.BlockSpec((64, d_h), lambda j: (0, 0)),
         pl.BlockSpec((d_h, 512), lambda j: (0, j)),
         pl.BlockSpec((1, 512), lambda j: (0, j))],
        pl.BlockSpec((64, 512), lambda j: (0, j)),
        jax.ShapeDtypeStruct((64, d_h), f32),
        h1, w2, b2)[:b, :20 * l]

    out_size = P["val_fc2"]["b"].shape[0]
    va = (P["val_fc1"]["w"].T, P["val_fc1"]["b"][None, :],
          P["val_fc2"]["w"].T, P["val_fc2"]["b"][None, :])
    val = _tc_call(
        _val_head_kernel, (1,),
        [pl.BlockSpec((b, l, 32), lambda i: (0, 0, 0))] + [_full(a) for a in va],
        pl.BlockSpec((b, out_size), lambda i: (0, 0)),
        jax.ShapeDtypeStruct((b, out_size), f32),
        val_pre.reshape(b, l, 32), *va)

    return pol, val


# speedup vs baseline: 1.1461x; 1.1461x over previous
import functools
import jax
import jax.numpy as jnp
from jax import lax
from jax.experimental import pallas as pl
from jax.experimental.pallas import tpu as pltpu
from jax.experimental.pallas import tpu_sc as plsc

SH = 128
VHC = 16
W = SH + 3 * VHC
SEW = 32
EBLK = 1280
NBLK = 1000
EBLK2 = 2000
GCH = 80
SCH = 128
EPAD = 20480


def _bd3(w):
    return jnp.kron(jnp.eye(3, dtype=w.dtype), w)


def _summ(h):
    return jnp.kron(jnp.ones((3, 1), dtype=jnp.float32), jnp.eye(h, dtype=jnp.float32))


def _ln_s(s, w, b):
    mu = jnp.mean(s, axis=-1, keepdims=True)
    var = jnp.mean((s - mu) * (s - mu), axis=-1, keepdims=True)
    return (s - mu) / jnp.sqrt(var + 1e-5) * w + b


def _dot(x, w):
    return jnp.dot(x, w, preferred_element_type=jnp.float32)


def _vnorm(vh, summ):
    return jnp.sqrt(jnp.clip(_dot(vh * vh, summ), 1e-8, None))


def _vgate(vo, summ):
    sig = jax.nn.sigmoid(_vnorm(vo, summ))
    return vo * jnp.concatenate([sig, sig, sig], axis=1)


def _node_embed_kernel(s_ref, v_ref, lnw, lnb, whb, summ, wss, wsn, wsb, wvb,
                       out_ref):
    s = _ln_s(s_ref[...], lnw[...], lnb[...])
    v = v_ref[...]
    vn = jnp.sqrt(jnp.sum(v * v, axis=-1, keepdims=True) / 3.0 + 1e-8)
    v = v / vn
    vh = _dot(v, whb[...])
    nrm = _vnorm(vh, summ[...])
    so = _dot(s, wss[...]) + _dot(nrm, wsn[...]) + wsb[...]
    vo = _dot(vh, wvb[...])
    out_ref[...] = jnp.concatenate([so, vo], axis=1)


def _edge_embed_kernel(s_ref, v_ref, lnw, lnb, wh00, wss, wsn, wsb, wv00,
                       so_ref, vo_ref):
    s = _ln_s(s_ref[...], lnw[...], lnb[...])
    v = v_ref[...]
    vn = jnp.sqrt(jnp.sum(v * v, axis=-1, keepdims=True) + 1e-8)
    v = v / vn
    vh = v * wh00[0, 0]
    nrm = jnp.sqrt(jnp.clip(jnp.sum(vh * vh, axis=-1, keepdims=True), 1e-8, None))
    so_ref[...] = _dot(s, wss[...]) + _dot(nrm, wsn[...]) + wsb[...]
    vo_ref[...] = vh * wv00[0, 0]


def _pack_bf16_pair(lo, hi):
    lob = lax.bitcast_convert_type(
        lo.astype(jnp.bfloat16).astype(jnp.float32), jnp.int32)
    hib = lax.bitcast_convert_type(
        hi.astype(jnp.bfloat16).astype(jnp.float32), jnp.int32)
    return lax.bitwise_or(lax.shift_right_logical(lob, 16),
                          lax.bitwise_and(hib, jnp.int32(-65536)))


def _unpack_bf16_pair(g32):
    lo = lax.bitcast_convert_type(lax.shift_left(g32, 16), jnp.float32)
    hi = lax.bitcast_convert_type(
        lax.bitwise_and(g32, jnp.int32(-65536)), jnp.float32)
    return lo, hi


def _node_pre_kernel(x_ref, wa, wsrc, wc, wdst, a_ref, c_ref):
    x = x_ref[...]
    s, v = x[:, :SH], x[:, SH:]
    blk = s.shape[0]
    pad = jnp.zeros((blk, SH - 99), jnp.float32)
    a_ref[...] = _pack_bf16_pair(
        _dot(s, wa[...]), jnp.concatenate([_dot(v, wsrc[...]), pad], 1))
    c_ref[...] = _pack_bf16_pair(
        _dot(s, wc[...]), jnp.concatenate([_dot(v, wdst[...]), pad], 1))


def _message_kernel(ga_ref, gc_ref, es_ref, ev_ref,
                    w0ev, summ33, ws0es, ws0n, ws0bias,
                    wv0b, summ16,
                    wh1b, ws1s, ws1n, ws1bias, wv1b,
                    wh2b, ws2s, ws2n, ws2bias, wv2b,
                    ms_ref, mv_ref):
    la, ha = _unpack_bf16_pair(ga_ref[...])
    lc, hc = _unpack_bf16_pair(gc_ref[...])
    es, ev = es_ref[...], ev_ref[...]
    vh0 = (ha + hc)[:, :99] + _dot(ev, w0ev[...])
    nrm0 = _vnorm(vh0, summ33[...])
    s0 = (la + lc + _dot(es, ws0es[...])
          + _dot(nrm0, ws0n[...]) + ws0bias[...])
    s0 = jnp.maximum(s0, 0.0)
    v0 = _vgate(_dot(vh0, wv0b[...]), summ16[...])
    vh1 = _dot(v0, wh1b[...])
    nrm1 = _vnorm(vh1, summ16[...])
    s1 = jnp.maximum(_dot(s0, ws1s[...]) + _dot(nrm1, ws1n[...]) + ws1bias[...], 0.0)
    v1 = _vgate(_dot(vh1, wv1b[...]), summ16[...])
    vh2 = _dot(v1, wh2b[...])
    nrm2 = _vnorm(vh2, summ16[...])
    s2 = _dot(s1, ws2s[...]) + _dot(nrm2, ws2n[...]) + ws2bias[...]
    v2 = _dot(vh2, wv2b[...])
    ms_ref[...] = s2
    mv_ref[...] = jnp.concatenate(
        [v2, jnp.zeros((v2.shape[0], 128 - 3 * VHC), jnp.float32)], axis=1)


def _cnt_kernel(c_ref, out_ref):
    c = c_ref[...]
    out_ref[...] = jnp.maximum(c[0][:, :16], 1.0)


def _node_update_kernel(x_ref, p_ref, c_ref,
                        ln0w, ln0b,
                        f0wh, summ32, f0wss, f0wsn, f0wsb, f0wv,
                        f1wh, f1wss, f1wsn, f1wsb, f1wv, summ16,
                        ln1w, ln1b,
                        out_ref):
    x = x_ref[...]
    cnt = c_ref[...][:, :1]
    p = p_ref[...]
    s = x[:, :SH] + p[0] / cnt
    v = x[:, SH:] + p[1][:, :3 * VHC] / cnt
    s = _ln_s(s, ln0w[...], ln0b[...])
    vn = jnp.sqrt(jnp.sum(v * v, axis=-1, keepdims=True) / VHC + 1e-8)
    v = v / vn
    vh = _dot(v, f0wh[...])
    nrm = _vnorm(vh, summ32[...])
    fs = jnp.maximum(_dot(s, f0wss[...]) + _dot(nrm, f0wsn[...]) + f0wsb[...], 0.0)
    fv = _vgate(_dot(vh, f0wv[...]), summ32[...])
    vh1 = _dot(fv, f1wh[...])
    nrm1 = _vnorm(vh1, summ32[...])
    fs1 = _dot(fs, f1wss[...]) + _dot(nrm1, f1wsn[...]) + f1wsb[...]
    fv1 = _dot(vh1, f1wv[...])
    s = s + fs1
    v = v + fv1
    s = _ln_s(s, ln1w[...], ln1b[...])
    vn = jnp.sqrt(jnp.sum(v * v, axis=-1, keepdims=True) / VHC + 1e-8)
    v = v / vn
    out_ref[...] = jnp.concatenate([s, v], axis=1)


def _heads_pre_kernel(x_ref, av_ref,
                      plnw, plnb, pwh, summ16, pwss, pwsn, pwsb,
                      vlnw, vlnb, vwh, vwss, vwsn, vwsb,
                      pol_ref, val_ref):
    x = x_ref[...]
    s, v = x[:, :SH], x[:, SH:]

    def head(lnw, lnb, whb, wss, wsn, wsb):
        s2 = _ln_s(s, lnw, lnb)
        vn = jnp.sqrt(jnp.sum(v * v, axis=-1, keepdims=True) / VHC + 1e-8)
        v2 = v / vn
        vh = _dot(v2, whb)
        nrm = _vnorm(vh, summ16[...])
        return _dot(s2, wss) + _dot(nrm, wsn) + wsb

    pol_ref[...] = head(plnw[...], plnb[...], pwh[...], pwss[...], pwsn[...],
                        pwsb[...]) * av_ref[...]
    val_ref[...] = head(vlnw[...], vlnb[...], vwh[...], vwss[...], vwsn[...],
                        vwsb[...])


def _matmul_bias_kernel(x_ref, w_ref, b_ref, out_ref, *, act):
    h = _dot(x_ref[...], w_ref[...]) + b_ref[...]
    if act:
        h = jnp.maximum(h, 0.0)
    out_ref[...] = h


def _val_head_kernel(vp_ref, w1, b1, w2, b2, out_ref):
    vsum = jnp.sum(vp_ref[...], axis=1)
    h = _dot(vsum, w1[...]) + b1[...]
    h = jnp.where(h > 0, h, 0.01 * h)
    out_ref[...] = _dot(h, w2[...]) + b2[...]


def _sc_gather2(tab_a, tab_c, idx_src, idx_dst):
    n, w = tab_a.shape
    dt = tab_a.dtype
    e = idx_src.shape[0]
    info = plsc.get_sparse_core_info()
    nw = info.num_cores * info.num_subcores
    per_w = e // nw
    n_ch = per_w // GCH
    mesh = plsc.VectorSubcoreMesh(core_axis_name="c", subcore_axis_name="s")

    @functools.partial(
        pl.kernel, mesh=mesh,
        out_type=[jax.ShapeDtypeStruct((e, w), dt),
                  jax.ShapeDtypeStruct((e, w), dt)],
        scratch_types=[pltpu.VMEM((per_w,), jnp.int32),
                       pltpu.VMEM((per_w,), jnp.int32),
                       pltpu.VMEM((GCH, w), dt),
                       pltpu.VMEM((GCH, w), dt),
                       pltpu.VMEM((GCH, w), dt),
                       pltpu.VMEM((GCH, w), dt),
                       pltpu.SemaphoreType.DMA, pltpu.SemaphoreType.DMA,
                       pltpu.SemaphoreType.DMA, pltpu.SemaphoreType.DMA,
                       pltpu.SemaphoreType.DMA, pltpu.SemaphoreType.DMA,
                       pltpu.SemaphoreType.DMA, pltpu.SemaphoreType.DMA],
    )
    def k(ta_h, tc_h, src_h, dst_h, oa_h, oc_h,
          isv, idv, ra0, ra1, rc0, rc1,
          sga0, sga1, sgc0, sgc1, swa0, swa1, swc0, swc1):
        wid = lax.axis_index("c") * info.num_subcores + lax.axis_index("s")
        base = wid * per_w
        ra, rc = (ra0, ra1), (rc0, rc1)
        sga, sgc = (sga0, sga1), (sgc0, sgc1)
        swa, swc = (swa0, swa1), (swc0, swc1)
        pltpu.sync_copy(src_h.at[pl.ds(base, per_w)], isv)
        pltpu.sync_copy(dst_h.at[pl.ds(base, per_w)], idv)

        def start_gather(i, p):
            off = i * GCH
            pltpu.async_copy(ta_h.at[isv.at[pl.ds(off, GCH)]], ra[p], sga[p])
            pltpu.async_copy(tc_h.at[idv.at[pl.ds(off, GCH)]], rc[p], sgc[p])

        def wait_writeouts(p):
            pltpu.make_async_copy(ra[p], oa_h.at[pl.ds(base, GCH)], swa[p]).wait()
            pltpu.make_async_copy(rc[p], oc_h.at[pl.ds(base, GCH)], swc[p]).wait()

        def body(i2, carry):
            i0 = i2 * 2
            for p in (0, 1):
                @pl.when(i2 >= 1)
                def _():
                    wait_writeouts(p)
                start_gather(i0 + p, p)
            for p in (0, 1):
                pltpu.make_async_copy(
                    ta_h.at[isv.at[pl.ds(0, GCH)]], ra[p], sga[p]).wait()
                pltpu.make_async_copy(
                    tc_h.at[idv.at[pl.ds(0, GCH)]], rc[p], sgc[p]).wait()
                off = base + (i0 + p) * GCH
                pltpu.async_copy(ra[p], oa_h.at[pl.ds(off, GCH)], swa[p])
                pltpu.async_copy(rc[p], oc_h.at[pl.ds(off, GCH)], swc[p])
            return carry

        lax.fori_loop(0, n_ch // 2, body, 0)
        for p in (0, 1):
            wait_writeouts(p)

    return k(tab_a, tab_c, idx_src, idx_dst)


def _sc_scatter_cols(ms, mv, idx, zeros_blk):
    e = ms.shape[0]
    info = plsc.get_sparse_core_info()
    ns = info.num_subcores
    per_t = e // ns
    n_ch = per_t // SCH
    rpt = zeros_blk.shape[0]
    npad = rpt * ns
    idx2d = idx.reshape(e // SCH, SCH)
    mesh = plsc.VectorSubcoreMesh(core_axis_name="c", subcore_axis_name="s")

    @functools.partial(
        pl.kernel, mesh=mesh,
        out_type=jax.ShapeDtypeStruct((2, npad, 128), jnp.float32),
        scratch_types=[pltpu.VMEM((n_ch, SCH), jnp.int32),
                       pltpu.VMEM((SCH, 128), jnp.float32),
                       pltpu.VMEM((SCH, 128), jnp.float32),
                       pltpu.VMEM_SHARED((npad, 128), jnp.float32),
                       pltpu.SemaphoreType.DMA, pltpu.SemaphoreType.DMA,
                       pltpu.SemaphoreType.DMA, pltpu.SemaphoreType.DMA],
    )
    def k(ms_h, mv_h, idx_h, zeros_h, out_h, idx_v, r0, r1, acc,
          src0, src1, ssa0, ssa1):
        c = lax.axis_index("c")
        s = lax.axis_index("s")
        rows = (r0, r1)
        src_sem = (src0, src1)
        sa_sem = (ssa0, ssa1)
        pltpu.sync_copy(zeros_h, acc.at[pl.ds(s * rpt, rpt)])
        pltpu.sync_copy(idx_h.at[pl.ds(s * n_ch, n_ch)], idx_v)
        plsc.subcore_barrier()

        def start_rowcopy(j, p):
            b = s * per_t + j * SCH

            @pl.when(c == 0)
            def _():
                pltpu.async_copy(ms_h.at[pl.ds(b, SCH)], rows[p], src_sem[p])

            @pl.when(c == 1)
            def _():
                pltpu.async_copy(mv_h.at[pl.ds(b, SCH)], rows[p], src_sem[p])

        def body(j2, carry):
            j0 = j2 * 2
            for p in (0, 1):
                @pl.when(j2 >= 1)
                def _():
                    pltpu.make_async_copy(
                        rows[p], acc.at[idx_v.at[0]], sa_sem[p]).wait()
                start_rowcopy(j0 + p, p)
            for p in (0, 1):
                pltpu.make_async_copy(
                    ms_h.at[pl.ds(0, SCH)], rows[p], src_sem[p]).wait()
                pltpu.async_copy(rows[p], acc.at[idx_v.at[j0 + p]],
                                 sa_sem[p], add=True)
            return carry

        lax.fori_loop(0, n_ch // 2, body, 0)
        for p in (0, 1):
            pltpu.make_async_copy(rows[p], acc.at[idx_v.at[0]], sa_sem[p]).wait()
        plsc.subcore_barrier()
        pltpu.sync_copy(acc.at[pl.ds(s * rpt, rpt)],
                        out_h.at[c, pl.ds(s * rpt, rpt)])

    return k(ms, mv, idx2d, zeros_blk)


def _sc_counts(idx, e, ones_blk, zeros_blk):
    info = plsc.get_sparse_core_info()
    ns = info.num_subcores
    n_ch = e // (ns * SCH)
    rpt = zeros_blk.shape[0]
    npad = rpt * ns
    mesh = plsc.VectorSubcoreMesh(core_axis_name="c", subcore_axis_name="s")

    @functools.partial(
        pl.kernel, mesh=mesh,
        out_type=jax.ShapeDtypeStruct((2, npad, 128), jnp.float32),
        scratch_types=[pltpu.VMEM((SCH,), jnp.int32),
                       pltpu.VMEM((SCH, 128), jnp.float32),
                       pltpu.VMEM_SHARED((npad, 128), jnp.float32),
                       pltpu.SemaphoreType.DMA],
    )
    def k(idx_h, ones_h, zeros_h, out_h, idx_v, rows_v, acc, sem):
        c = lax.axis_index("c")
        s = lax.axis_index("s")
        pltpu.sync_copy(zeros_h, acc.at[pl.ds(s * rpt, rpt)])
        pltpu.sync_copy(ones_h, rows_v)
        plsc.subcore_barrier()

        def body(i, carry):
            b = (i * ns + s) * SCH
            pltpu.sync_copy(idx_h.at[pl.ds(b, SCH)], idx_v)
            pltpu.sync_copy(rows_v, acc.at[idx_v], add=True)
            return carry

        lax.fori_loop(0, n_ch, body, 0)
        plsc.subcore_barrier()
        pltpu.sync_copy(acc.at[pl.ds(s * rpt, rpt)],
                        out_h.at[c, pl.ds(s * rpt, rpt)])

    return k(idx, ones_blk, zeros_blk)


def _dbg_gather(ta, tc, i_s, i_d):
    return ta[i_s], tc[i_d]


def _dbg_scatter(ms, mv, idx, z):
    npad = z.shape[0] * 16
    s0 = jax.ops.segment_sum(ms, idx, num_segments=npad)
    s1 = jax.ops.segment_sum(mv, idx, num_segments=npad)
    return jnp.stack([s0, s1])


def _dbg_counts(idx, e, ones, z):
    npad = z.shape[0] * 16
    c = jax.ops.segment_sum(jnp.ones((e,), jnp.float32), idx, num_segments=npad)
    c = jnp.broadcast_to(c[:, None], (npad, 128))
    return jnp.stack([c, c])


def _tc_call(body, grid, in_specs, out_specs, out_shape, *args):
    return pl.pallas_call(
        body,
        grid=grid,
        in_specs=in_specs,
        out_specs=out_specs,
        out_shape=out_shape,
    )(*args)


def _full(a):
    return pl.BlockSpec(a.shape, lambda i: tuple(0 for _ in a.shape))


def kernel(node_s, node_v, edge_s, edge_v, avaliable_pos, params, edge_index,
           batch_ids, ptr):
    f32 = jnp.float32
    n = node_s.shape[0]
    e = edge_s.shape[0]
    b = ptr.shape[0] - 1
    l = n // b
    src = edge_index[0].astype(jnp.int32)
    dst = edge_index[1].astype(jnp.int32)
    ep = ((e + EPAD - 1) // EPAD) * EPAD
    rpt = (n // 16 // 8 + 1) * 8
    npad = rpt * 16
    pad = ep - e
    src_g = jnp.concatenate([src, jnp.zeros((pad,), jnp.int32)])
    dst_g = jnp.concatenate([dst, jnp.zeros((pad,), jnp.int32)])
    dst_s = jnp.concatenate([dst, jnp.full((pad,), npad - 1, jnp.int32)])

    P = params
    su16, su32, su33 = _summ(16), _summ(32), _summ(33)

    ng = P["node_gvp"]
    ne_args = (P["node_ln"]["w"][None, :], P["node_ln"]["b"][None, :],
               _bd3(ng["wh"].T), su16,
               ng["ws"]["w"].T[:SH], ng["ws"]["w"].T[SH:],
               ng["ws"]["b"][None, :], _bd3(ng["wv"].T))
    eg = P["edge_gvp"]
    ee_args = (P["edge_ln"]["w"][None, :], P["edge_ln"]["b"][None, :],
               eg["wh"], eg["ws"]["w"].T[:SEW], eg["ws"]["w"].T[SEW:],
               eg["ws"]["b"][None, :], eg["wv"])

    def pre_args(cp):
        m0 = cp["m0"]
        wh0t = m0["wh"].T
        ws0t = m0["ws"]["w"].T
        return (ws0t[0:SH], _bd3(wh0t[0:16]),
                ws0t[SH + SEW:2 * SH + SEW], _bd3(wh0t[17:33]))

    def msg_args(cp):
        m0, m1, m2 = cp["m0"], cp["m1"], cp["m2"]
        wh0t = m0["wh"].T
        ws0t = m0["ws"]["w"].T
        return (_bd3(wh0t[16:17]), su33,
                ws0t[SH:SH + SEW],
                ws0t[2 * SH + SEW:], m0["ws"]["b"][None, :],
                _bd3(m0["wv"].T), su16,
                _bd3(m1["wh"].T), m1["ws"]["w"].T[:SH], m1["ws"]["w"].T[SH:],
                m1["ws"]["b"][None, :], _bd3(m1["wv"].T),
                _bd3(m2["wh"].T), m2["ws"]["w"].T[:SH], m2["ws"]["w"].T[SH:],
                m2["ws"]["b"][None, :], _bd3(m2["wv"].T))

    def upd_args(lp):
        f0, f1 = lp["ff0"], lp["ff1"]
        return (lp["ln0"]["w"][None, :], lp["ln0"]["b"][None, :],
                _bd3(f0["wh"].T), su32,
                f0["ws"]["w"].T[:SH], f0["ws"]["w"].T[SH:],
                f0["ws"]["b"][None, :], _bd3(f0["wv"].T),
                _bd3(f1["wh"].T), f1["ws"]["w"].T[:4 * SH],
                f1["ws"]["w"].T[4 * SH:], f1["ws"]["b"][None, :],
                _bd3(f1["wv"].T), su16,
                lp["ln1"]["w"][None, :], lp["ln1"]["b"][None, :])

    pg, vg = P["pol_gvp"], P["val_gvp"]
    hp_args = (P["pol_ln"]["w"][None, :], P["pol_ln"]["b"][None, :],
               _bd3(pg["wh"].T), su16,
               pg["ws"]["w"].T[:SH], pg["ws"]["w"].T[SH:],
               pg["ws"]["b"][None, :],
               P["val_ln"]["w"][None, :], P["val_ln"]["b"][None, :],
               _bd3(vg["wh"].T),
               vg["ws"]["w"].T[:SH], vg["ws"]["w"].T[SH:],
               vg["ws"]["b"][None, :])

    nv_flat = node_v.swapaxes(1, 2).reshape(n, 9)
    ngrid = n // NBLK
    nspec = pl.BlockSpec((NBLK, W), lambda i: (i, 0))
    x = _tc_call(_node_embed_kernel, (ngrid,),
                 [pl.BlockSpec((NBLK, SH), lambda i: (i, 0)),
                  pl.BlockSpec((NBLK, 9), lambda i: (i, 0))]
                 + [_full(a) for a in ne_args],
                 nspec, jax.ShapeDtypeStruct((n, W), f32),
                 node_s, nv_flat, *ne_args)

    ev_flat = edge_v.swapaxes(1, 2).reshape(e, 3)
    egrid_e = e // EBLK
    egrid = ep // EBLK
    emax = egrid_e - 1
    es2, ev2 = _tc_call(
        _edge_embed_kernel, (egrid_e,),
        [pl.BlockSpec((EBLK, SEW), lambda i: (i, 0)),
         pl.BlockSpec((EBLK, 3), lambda i: (i, 0))]
        + [_full(a) for a in ee_args],
        [pl.BlockSpec((EBLK, SEW), lambda i: (i, 0)),
         pl.BlockSpec((EBLK, 3), lambda i: (i, 0))],
        [jax.ShapeDtypeStruct((e, SEW), f32), jax.ShapeDtypeStruct((e, 3), f32)],
        edge_s, ev_flat, *ee_args)

    ones_ch = jnp.ones((SCH, 128), f32)
    zeros128 = jnp.zeros((rpt, 128), f32)
    cnt_parts = _sc_counts(dst_s, ep, ones_ch, zeros128)
    cnt16 = _tc_call(
        _cnt_kernel, (ngrid,),
        [pl.BlockSpec((2, NBLK, 128), lambda i: (0, i, 0))],
        pl.BlockSpec((NBLK, 16), lambda i: (i, 0)),
        jax.ShapeDtypeStruct((n, 16), f32),
        cnt_parts)
    espec = pl.BlockSpec((EBLK, W), lambda i: (i, 0))

    p256 = pl.BlockSpec((NBLK, 128), lambda i: (i, 0))
    g256 = pl.BlockSpec((EBLK, 128), lambda i: (i, 0))
    for li in range(3):
        lp = P["layer%d" % li]
        pargs = pre_args(lp["conv"])
        a_t, c_t = _tc_call(
            _node_pre_kernel, (ngrid,),
            [nspec] + [_full(w) for w in pargs],
            [p256, p256],
            [jax.ShapeDtypeStruct((n, 128), jnp.int32),
             jax.ShapeDtypeStruct((n, 128), jnp.int32)],
            x, *pargs)
        ga, gc = _sc_gather2(a_t, c_t, src_g, dst_g)
        margs = msg_args(lp["conv"])
        ms, mv = _tc_call(
            _message_kernel, (egrid,),
            [g256, g256,
             pl.BlockSpec((EBLK, SEW), lambda i: (jnp.minimum(i, emax), 0)),
             pl.BlockSpec((EBLK, 3), lambda i: (jnp.minimum(i, emax), 0))]
            + [_full(a) for a in margs],
            [pl.BlockSpec((EBLK, SH), lambda i: (i, 0)),
             pl.BlockSpec((EBLK, 128), lambda i: (i, 0))],
            [jax.ShapeDtypeStruct((ep, SH), f32),
             jax.ShapeDtypeStruct((ep, 128), f32)],
            ga, gc, es2, ev2, *margs)
        parts = _sc_scatter_cols(ms, mv, dst_s, zeros128)
        uargs = upd_args(lp)
        x = _tc_call(
            _node_update_kernel, (ngrid,),
            [nspec,
             pl.BlockSpec((2, NBLK, 128), lambda i: (0, i, 0)),
             pl.BlockSpec((NBLK, 16), lambda i: (i, 0))]
            + [_full(a) for a in uargs],
            nspec, jax.ShapeDtypeStruct((n, W), f32),
            x, parts, cnt16, *uargs)

    av = avaliable_pos.reshape(n, 1)
    pol_pre, val_pre = _tc_call(
        _heads_pre_kernel, (ngrid,),
        [nspec, pl.BlockSpec((NBLK, 1), lambda i: (i, 0))]
        + [_full(a) for a in hp_args],
        [pl.BlockSpec((NBLK, 32), lambda i: (i, 0)),
         pl.BlockSpec((NBLK, 32), lambda i: (i, 0))],
        [jax.ShapeDtypeStruct((n, 32), f32), jax.ShapeDtypeStruct((n, 32), f32)],
        x, av, *hp_args)

    d_in = l * 32
    d_h = pl.cdiv(20 * l, 512) * 512
    xp = jnp.zeros((64, d_in), f32).at[:b].set(pol_pre.reshape(b, d_in))
    w1 = P["pol_fc1"]["w"].T
    w1 = jnp.zeros((d_in, d_h), f32).at[:, :20 * l].set(w1)
    b1 = jnp.zeros((1, d_h), f32).at[0, :20 * l].set(P["pol_fc1"]["b"])
    w2 = jnp.zeros((d_h, d_h), f32).at[:20 * l, :20 * l].set(P["pol_fc2"]["w"].T)
    b2 = jnp.zeros((1, d_h), f32).at[0, :20 * l].set(P["pol_fc2"]["b"])

    hgrid = d_h // 512
    h1 = _tc_call(
        functools.partial(_matmul_bias_kernel, act=True), (hgrid,),
        [pl.BlockSpec((64, d_in), lambda j: (0, 0)),
         pl.BlockSpec((d_in, 512), lambda j: (0, j)),
         pl.BlockSpec((1, 512), lambda j: (0, j))],
        pl.BlockSpec((64, 512), lambda j: (0, j)),
        jax.ShapeDtypeStruct((64, d_h), f32),
        xp, w1, b1)
    pol = _tc_call(
        functools.partial(_matmul_bias_kernel, act=False), (hgrid,),
        [pl.BlockSpec((64, d_h), lambda j: (0, 0)),
         pl.BlockSpec((d_h, 512), lambda j: (0, j)),
         pl.BlockSpec((1, 512), lambda j: (0, j))],
        pl.BlockSpec((64, 512), lambda j: (0, j)),
        jax.ShapeDtypeStruct((64, d_h), f32),
        h1, w2, b2)[:b, :20 * l]

    out_size = P["val_fc2"]["b"].shape[0]
    va = (P["val_fc1"]["w"].T, P["val_fc1"]["b"][None, :],
          P["val_fc2"]["w"].T, P["val_fc2"]["b"][None, :])
    val = _tc_call(
        _val_head_kernel, (1,),
        [pl.BlockSpec((b, l, 32), lambda i: (0, 0, 0))] + [_full(a) for a in va],
        pl.BlockSpec((b, out_size), lambda i: (0, 0)),
        jax.ShapeDtypeStruct((b, out_size), f32),
        val_pre.reshape(b, l, 32), *va)

    return pol, val


# speedup vs baseline: 1.1948x; 1.0424x over previous
import functools
import jax
import jax.numpy as jnp
from jax import lax
from jax.experimental import pallas as pl
from jax.experimental.pallas import tpu as pltpu
from jax.experimental.pallas import tpu_sc as plsc

SH = 128
VHC = 16
W = SH + 3 * VHC
SEW = 32
EBLK = 1280
NBLK = 1000
EBLK2 = 2000
GCH = 80
SCH = 128
EPAD = 20480


def _bd3(w):
    return jnp.kron(jnp.eye(3, dtype=w.dtype), w)


def _summ(h):
    return jnp.kron(jnp.ones((3, 1), dtype=jnp.float32), jnp.eye(h, dtype=jnp.float32))


def _ln_s(s, w, b):
    mu = jnp.mean(s, axis=-1, keepdims=True)
    var = jnp.mean((s - mu) * (s - mu), axis=-1, keepdims=True)
    return (s - mu) / jnp.sqrt(var + 1e-5) * w + b


def _dot(x, w):
    return jnp.dot(x, w, preferred_element_type=jnp.float32)


def _vnorm(vh, summ):
    return jnp.sqrt(jnp.clip(_dot(vh * vh, summ), 1e-8, None))


def _vgate(vo, summ):
    sig = jax.nn.sigmoid(_vnorm(vo, summ))
    return vo * jnp.concatenate([sig, sig, sig], axis=1)


def _node_embed_kernel(s_ref, v_ref, lnw, lnb, whb, summ, wss, wsn, wsb, wvb,
                       out_ref):
    s = _ln_s(s_ref[...], lnw[...], lnb[...])
    v = v_ref[...]
    vn = jnp.sqrt(jnp.sum(v * v, axis=-1, keepdims=True) / 3.0 + 1e-8)
    v = v / vn
    vh = _dot(v, whb[...])
    nrm = _vnorm(vh, summ[...])
    so = _dot(s, wss[...]) + _dot(nrm, wsn[...]) + wsb[...]
    vo = _dot(vh, wvb[...])
    out_ref[...] = jnp.concatenate([so, vo], axis=1)


def _edge_embed_kernel(s_ref, v_ref, lnw, lnb, wh00, wss, wsn, wsb, wv00,
                       so_ref, vo_ref):
    s = _ln_s(s_ref[...], lnw[...], lnb[...])
    v = v_ref[...]
    vn = jnp.sqrt(jnp.sum(v * v, axis=-1, keepdims=True) + 1e-8)
    v = v / vn
    vh = v * wh00[0, 0]
    nrm = jnp.sqrt(jnp.clip(jnp.sum(vh * vh, axis=-1, keepdims=True), 1e-8, None))
    so_ref[...] = _dot(s, wss[...]) + _dot(nrm, wsn[...]) + wsb[...]
    vo_ref[...] = vh * wv00[0, 0]


def _pack_bf16_pair(lo, hi):
    lob = lax.bitcast_convert_type(
        lo.astype(jnp.bfloat16).astype(jnp.float32), jnp.int32)
    hib = lax.bitcast_convert_type(
        hi.astype(jnp.bfloat16).astype(jnp.float32), jnp.int32)
    return lax.bitwise_or(lax.shift_right_logical(lob, 16),
                          lax.bitwise_and(hib, jnp.int32(-65536)))


def _unpack_bf16_pair(g32):
    lo = lax.bitcast_convert_type(lax.shift_left(g32, 16), jnp.float32)
    hi = lax.bitcast_convert_type(
        lax.bitwise_and(g32, jnp.int32(-65536)), jnp.float32)
    return lo, hi


def _node_pre_kernel(x_ref, wa, wsrc, wc, wdst, a_ref, c_ref):
    x = x_ref[...]
    s, v = x[:, :SH], x[:, SH:]
    blk = s.shape[0]
    pad = jnp.zeros((blk, SH - 99), jnp.float32)
    a_ref[...] = _pack_bf16_pair(
        _dot(s, wa[...]), jnp.concatenate([_dot(v, wsrc[...]), pad], 1))
    c_ref[...] = _pack_bf16_pair(
        _dot(s, wc[...]), jnp.concatenate([_dot(v, wdst[...]), pad], 1))


def _message_kernel(ga_ref, gc_ref, es_ref, ev_ref,
                    w0ev, summ33, ws0es, ws0n, ws0bias,
                    wv0b, summ16,
                    wh1b, ws1s, ws1n, ws1bias, wv1b,
                    wh2b, ws2s, ws2n, ws2bias, wv2b,
                    ms_ref, mv_ref):
    la, ha = _unpack_bf16_pair(ga_ref[...])
    lc, hc = _unpack_bf16_pair(gc_ref[...])
    es, ev = es_ref[...], ev_ref[...]
    vh0 = (ha + hc)[:, :99] + _dot(ev, w0ev[...])
    nrm0 = _vnorm(vh0, summ33[...])
    s0 = (la + lc + _dot(es, ws0es[...])
          + _dot(nrm0, ws0n[...]) + ws0bias[...])
    s0 = jnp.maximum(s0, 0.0)
    v0 = _vgate(_dot(vh0, wv0b[...]), summ16[...])
    vh1 = _dot(v0, wh1b[...])
    nrm1 = _vnorm(vh1, summ16[...])
    s1 = jnp.maximum(_dot(s0, ws1s[...]) + _dot(nrm1, ws1n[...]) + ws1bias[...], 0.0)
    v1 = _vgate(_dot(vh1, wv1b[...]), summ16[...])
    vh2 = _dot(v1, wh2b[...])
    nrm2 = _vnorm(vh2, summ16[...])
    s2 = _dot(s1, ws2s[...]) + _dot(nrm2, ws2n[...]) + ws2bias[...]
    v2 = _dot(vh2, wv2b[...])
    ms_ref[...] = s2
    mv_ref[...] = jnp.concatenate(
        [v2, jnp.zeros((v2.shape[0], 128 - 3 * VHC), jnp.float32)], axis=1)


def _cnt_kernel(c_ref, out_ref):
    c = c_ref[...]
    out_ref[...] = jnp.maximum(c[0][:, :16], 1.0)


def _node_update_kernel(x_ref, p_ref, c_ref,
                        ln0w, ln0b,
                        f0wh, summ32, f0wss, f0wsn, f0wsb, f0wv,
                        f1wh, f1wss, f1wsn, f1wsb, f1wv, summ16,
                        ln1w, ln1b,
                        out_ref):
    x = x_ref[...]
    cnt = c_ref[...][:, :1]
    p = p_ref[...]
    s = x[:, :SH] + p[0] / cnt
    v = x[:, SH:] + p[1][:, :3 * VHC] / cnt
    s = _ln_s(s, ln0w[...], ln0b[...])
    vn = jnp.sqrt(jnp.sum(v * v, axis=-1, keepdims=True) / VHC + 1e-8)
    v = v / vn
    vh = _dot(v, f0wh[...])
    nrm = _vnorm(vh, summ32[...])
    fs = jnp.maximum(_dot(s, f0wss[...]) + _dot(nrm, f0wsn[...]) + f0wsb[...], 0.0)
    fv = _vgate(_dot(vh, f0wv[...]), summ32[...])
    vh1 = _dot(fv, f1wh[...])
    nrm1 = _vnorm(vh1, summ32[...])
    fs1 = _dot(fs, f1wss[...]) + _dot(nrm1, f1wsn[...]) + f1wsb[...]
    fv1 = _dot(vh1, f1wv[...])
    s = s + fs1
    v = v + fv1
    s = _ln_s(s, ln1w[...], ln1b[...])
    vn = jnp.sqrt(jnp.sum(v * v, axis=-1, keepdims=True) / VHC + 1e-8)
    v = v / vn
    out_ref[...] = jnp.concatenate([s, v], axis=1)


def _heads_pre_kernel(x_ref, av_ref,
                      plnw, plnb, pwh, summ16, pwss, pwsn, pwsb,
                      vlnw, vlnb, vwh, vwss, vwsn, vwsb,
                      pol_ref, val_ref):
    x = x_ref[...]
    s, v = x[:, :SH], x[:, SH:]

    def head(lnw, lnb, whb, wss, wsn, wsb):
        s2 = _ln_s(s, lnw, lnb)
        vn = jnp.sqrt(jnp.sum(v * v, axis=-1, keepdims=True) / VHC + 1e-8)
        v2 = v / vn
        vh = _dot(v2, whb)
        nrm = _vnorm(vh, summ16[...])
        return _dot(s2, wss) + _dot(nrm, wsn) + wsb

    pol_ref[...] = head(plnw[...], plnb[...], pwh[...], pwss[...], pwsn[...],
                        pwsb[...]) * av_ref[...]
    val_ref[...] = head(vlnw[...], vlnb[...], vwh[...], vwss[...], vwsn[...],
                        vwsb[...])


def _matmul_bias_kernel(x_ref, w_ref, b_ref, out_ref, *, act):
    h = _dot(x_ref[...], w_ref[...]) + b_ref[...]
    if act:
        h = jnp.maximum(h, 0.0)
    out_ref[...] = h


def _val_head_kernel(vp_ref, w1, b1, w2, b2, out_ref):
    vsum = jnp.sum(vp_ref[...], axis=1)
    h = _dot(vsum, w1[...]) + b1[...]
    h = jnp.where(h > 0, h, 0.01 * h)
    out_ref[...] = _dot(h, w2[...]) + b2[...]


def _sc_gather2(tab_a, tab_c, idx_src, idx_dst):
    n, w = tab_a.shape
    dt = tab_a.dtype
    e = idx_src.shape[0]
    info = plsc.get_sparse_core_info()
    nw = info.num_cores * info.num_subcores
    per_w = e // nw
    n_ch = per_w // GCH
    mesh = plsc.VectorSubcoreMesh(core_axis_name="c", subcore_axis_name="s")

    @functools.partial(
        pl.kernel, mesh=mesh,
        out_type=[jax.ShapeDtypeStruct((e, w), dt),
                  jax.ShapeDtypeStruct((e, w), dt)],
        scratch_types=[pltpu.VMEM((per_w,), jnp.int32),
                       pltpu.VMEM((per_w,), jnp.int32),
                       pltpu.VMEM((GCH, w), dt),
                       pltpu.VMEM((GCH, w), dt),
                       pltpu.VMEM((GCH, w), dt),
                       pltpu.VMEM((GCH, w), dt),
                       pltpu.SemaphoreType.DMA, pltpu.SemaphoreType.DMA,
                       pltpu.SemaphoreType.DMA, pltpu.SemaphoreType.DMA,
                       pltpu.SemaphoreType.DMA, pltpu.SemaphoreType.DMA,
                       pltpu.SemaphoreType.DMA, pltpu.SemaphoreType.DMA],
    )
    def k(ta_h, tc_h, src_h, dst_h, oa_h, oc_h,
          isv, idv, ra0, ra1, rc0, rc1,
          sga0, sga1, sgc0, sgc1, swa0, swa1, swc0, swc1):
        wid = lax.axis_index("c") * info.num_subcores + lax.axis_index("s")
        base = wid * per_w
        ra, rc = (ra0, ra1), (rc0, rc1)
        sga, sgc = (sga0, sga1), (sgc0, sgc1)
        swa, swc = (swa0, swa1), (swc0, swc1)
        pltpu.sync_copy(src_h.at[pl.ds(base, per_w)], isv)
        pltpu.sync_copy(dst_h.at[pl.ds(base, per_w)], idv)

        def start_gather(i, p):
            off = i * GCH
            pltpu.async_copy(ta_h.at[isv.at[pl.ds(off, GCH)]], ra[p], sga[p])
            pltpu.async_copy(tc_h.at[idv.at[pl.ds(off, GCH)]], rc[p], sgc[p])

        def wait_writeouts(p):
            pltpu.make_async_copy(ra[p], oa_h.at[pl.ds(base, GCH)], swa[p]).wait()
            pltpu.make_async_copy(rc[p], oc_h.at[pl.ds(base, GCH)], swc[p]).wait()

        def body(i2, carry):
            i0 = i2 * 2
            for p in (0, 1):
                @pl.when(i2 >= 1)
                def _():
                    wait_writeouts(p)
                start_gather(i0 + p, p)
            for p in (0, 1):
                pltpu.make_async_copy(
                    ta_h.at[isv.at[pl.ds(0, GCH)]], ra[p], sga[p]).wait()
                pltpu.make_async_copy(
                    tc_h.at[idv.at[pl.ds(0, GCH)]], rc[p], sgc[p]).wait()
                off = base + (i0 + p) * GCH
                pltpu.async_copy(ra[p], oa_h.at[pl.ds(off, GCH)], swa[p])
                pltpu.async_copy(rc[p], oc_h.at[pl.ds(off, GCH)], swc[p])
            return carry

        lax.fori_loop(0, n_ch // 2, body, 0)
        for p in (0, 1):
            wait_writeouts(p)

    return k(tab_a, tab_c, idx_src, idx_dst)


def _sc_scatter_cols(ms, mv, idx, zeros_blk):
    e = ms.shape[0]
    info = plsc.get_sparse_core_info()
    ns = info.num_subcores
    per_t = e // ns
    n_ch = per_t // SCH
    rpt = zeros_blk.shape[0]
    npad = rpt * ns
    idx2d = idx.reshape(e // SCH, SCH)
    mesh = plsc.VectorSubcoreMesh(core_axis_name="c", subcore_axis_name="s")

    @functools.partial(
        pl.kernel, mesh=mesh,
        out_type=jax.ShapeDtypeStruct((2, npad, 128), jnp.float32),
        scratch_types=[pltpu.VMEM((n_ch, SCH), jnp.int32),
                       pltpu.VMEM((SCH, 128), jnp.float32),
                       pltpu.VMEM((SCH, 128), jnp.float32),
                       pltpu.VMEM_SHARED((npad, 128), jnp.float32),
                       pltpu.SemaphoreType.DMA, pltpu.SemaphoreType.DMA,
                       pltpu.SemaphoreType.DMA, pltpu.SemaphoreType.DMA],
    )
    def k(ms_h, mv_h, idx_h, zeros_h, out_h, idx_v, r0, r1, acc,
          src0, src1, ssa0, ssa1):
        c = lax.axis_index("c")
        s = lax.axis_index("s")
        rows = (r0, r1)
        src_sem = (src0, src1)
        sa_sem = (ssa0, ssa1)
        pltpu.sync_copy(zeros_h, acc.at[pl.ds(s * rpt, rpt)])
        pltpu.sync_copy(idx_h.at[pl.ds(s * n_ch, n_ch)], idx_v)
        plsc.subcore_barrier()

        def start_rowcopy(j, p):
            b = s * per_t + j * SCH

            @pl.when(c == 0)
            def _():
                pltpu.async_copy(ms_h.at[pl.ds(b, SCH)], rows[p], src_sem[p])

            @pl.when(c == 1)
            def _():
                pltpu.async_copy(mv_h.at[pl.ds(b, SCH)], rows[p], src_sem[p])

        def body(j2, carry):
            j0 = j2 * 2
            for p in (0, 1):
                @pl.when(j2 >= 1)
                def _():
                    pltpu.make_async_copy(
                        rows[p], acc.at[idx_v.at[0]], sa_sem[p]).wait()
                start_rowcopy(j0 + p, p)
            for p in (0, 1):
                pltpu.make_async_copy(
                    ms_h.at[pl.ds(0, SCH)], rows[p], src_sem[p]).wait()
                pltpu.async_copy(rows[p], acc.at[idx_v.at[j0 + p]],
                                 sa_sem[p], add=True)
            return carry

        lax.fori_loop(0, n_ch // 2, body, 0)
        for p in (0, 1):
            pltpu.make_async_copy(rows[p], acc.at[idx_v.at[0]], sa_sem[p]).wait()
        plsc.subcore_barrier()
        pltpu.sync_copy(acc.at[pl.ds(s * rpt, rpt)],
                        out_h.at[c, pl.ds(s * rpt, rpt)])

    return k(ms, mv, idx2d, zeros_blk)


def _sc_counts(idx, e, ones_blk, zeros_blk):
    info = plsc.get_sparse_core_info()
    ns = info.num_subcores
    n_ch = e // (ns * SCH)
    rpt = zeros_blk.shape[0]
    npad = rpt * ns
    mesh = plsc.VectorSubcoreMesh(core_axis_name="c", subcore_axis_name="s")

    @functools.partial(
        pl.kernel, mesh=mesh,
        out_type=jax.ShapeDtypeStruct((2, npad, 128), jnp.float32),
        scratch_types=[pltpu.VMEM((SCH,), jnp.int32),
                       pltpu.VMEM((SCH, 128), jnp.float32),
                       pltpu.VMEM_SHARED((npad, 128), jnp.float32),
                       pltpu.SemaphoreType.DMA],
    )
    def k(idx_h, ones_h, zeros_h, out_h, idx_v, rows_v, acc, sem):
        c = lax.axis_index("c")
        s = lax.axis_index("s")
        pltpu.sync_copy(zeros_h, acc.at[pl.ds(s * rpt, rpt)])
        pltpu.sync_copy(ones_h, rows_v)
        plsc.subcore_barrier()

        def body(i, carry):
            b = (i * ns + s) * SCH
            pltpu.sync_copy(idx_h.at[pl.ds(b, SCH)], idx_v)
            pltpu.sync_copy(rows_v, acc.at[idx_v], add=True)
            return carry

        lax.fori_loop(0, n_ch, body, 0)
        plsc.subcore_barrier()
        pltpu.sync_copy(acc.at[pl.ds(s * rpt, rpt)],
                        out_h.at[c, pl.ds(s * rpt, rpt)])

    return k(idx, ones_blk, zeros_blk)


def _dbg_gather(ta, tc, i_s, i_d):
    return ta[i_s], tc[i_d]


def _dbg_scatter(ms, mv, idx, z):
    npad = z.shape[0] * 16
    s0 = jax.ops.segment_sum(ms, idx, num_segments=npad)
    s1 = jax.ops.segment_sum(mv, idx, num_segments=npad)
    return jnp.stack([s0, s1])


def _dbg_counts(idx, e, ones, z):
    npad = z.shape[0] * 16
    c = jax.ops.segment_sum(jnp.ones((e,), jnp.float32), idx, num_segments=npad)
    c = jnp.broadcast_to(c[:, None], (npad, 128))
    return jnp.stack([c, c])


def _tc_call(body, grid, in_specs, out_specs, out_shape, *args):
    return pl.pallas_call(
        body,
        grid=grid,
        in_specs=in_specs,
        out_specs=out_specs,
        out_shape=out_shape,
    )(*args)


def _full(a):
    return pl.BlockSpec(a.shape, lambda i: tuple(0 for _ in a.shape))


def kernel(node_s, node_v, edge_s, edge_v, avaliable_pos, params, edge_index,
           batch_ids, ptr):
    f32 = jnp.float32
    n = node_s.shape[0]
    e = edge_s.shape[0]
    b = ptr.shape[0] - 1
    l = n // b
    src = edge_index[0].astype(jnp.int32)
    dst = edge_index[1].astype(jnp.int32)
    ep = ((e + EPAD - 1) // EPAD) * EPAD
    rpt = (n // 16 // 8 + 1) * 8
    npad = rpt * 16
    pad = ep - e
    src_g = jnp.concatenate([src, jnp.zeros((pad,), jnp.int32)])
    dst_g = jnp.concatenate([dst, jnp.zeros((pad,), jnp.int32)])
    dst_s = jnp.concatenate([dst, jnp.full((pad,), npad - 1, jnp.int32)])

    P = params
    su16, su32, su33 = _summ(16), _summ(32), _summ(33)

    ng = P["node_gvp"]
    ne_args = (P["node_ln"]["w"][None, :], P["node_ln"]["b"][None, :],
               _bd3(ng["wh"].T), su16,
               ng["ws"]["w"].T[:SH], ng["ws"]["w"].T[SH:],
               ng["ws"]["b"][None, :], _bd3(ng["wv"].T))
    eg = P["edge_gvp"]
    ee_args = (P["edge_ln"]["w"][None, :], P["edge_ln"]["b"][None, :],
               eg["wh"], eg["ws"]["w"].T[:SEW], eg["ws"]["w"].T[SEW:],
               eg["ws"]["b"][None, :], eg["wv"])

    def pre_args(cp):
        m0 = cp["m0"]
        wh0t = m0["wh"].T
        ws0t = m0["ws"]["w"].T
        return (ws0t[0:SH], _bd3(wh0t[0:16]),
                ws0t[SH + SEW:2 * SH + SEW], _bd3(wh0t[17:33]))

    def msg_args(cp):
        m0, m1, m2 = cp["m0"], cp["m1"], cp["m2"]
        wh0t = m0["wh"].T
        ws0t = m0["ws"]["w"].T
        return (_bd3(wh0t[16:17]), su33,
                ws0t[SH:SH + SEW],
                ws0t[2 * SH + SEW:], m0["ws"]["b"][None, :],
                _bd3(m0["wv"].T), su16,
                _bd3(m1["wh"].T), m1["ws"]["w"].T[:SH], m1["ws"]["w"].T[SH:],
                m1["ws"]["b"][None, :], _bd3(m1["wv"].T),
                _bd3(m2["wh"].T), m2["ws"]["w"].T[:SH], m2["ws"]["w"].T[SH:],
                m2["ws"]["b"][None, :], _bd3(m2["wv"].T))

    def upd_args(lp):
        f0, f1 = lp["ff0"], lp["ff1"]
        return (lp["ln0"]["w"][None, :], lp["ln0"]["b"][None, :],
                _bd3(f0["wh"].T), su32,
                f0["ws"]["w"].T[:SH], f0["ws"]["w"].T[SH:],
                f0["ws"]["b"][None, :], _bd3(f0["wv"].T),
                _bd3(f1["wh"].T), f1["ws"]["w"].T[:4 * SH],
                f1["ws"]["w"].T[4 * SH:], f1["ws"]["b"][None, :],
                _bd3(f1["wv"].T), su16,
                lp["ln1"]["w"][None, :], lp["ln1"]["b"][None, :])

    pg, vg = P["pol_gvp"], P["val_gvp"]
    hp_args = (P["pol_ln"]["w"][None, :], P["pol_ln"]["b"][None, :],
               _bd3(pg["wh"].T), su16,
               pg["ws"]["w"].T[:SH], pg["ws"]["w"].T[SH:],
               pg["ws"]["b"][None, :],
               P["val_ln"]["w"][None, :], P["val_ln"]["b"][None, :],
               _bd3(vg["wh"].T),
               vg["ws"]["w"].T[:SH], vg["ws"]["w"].T[SH:],
               vg["ws"]["b"][None, :])

    nv_flat = node_v.swapaxes(1, 2).reshape(n, 9)
    ngrid = n // NBLK
    nspec = pl.BlockSpec((NBLK, W), lambda i: (i, 0))
    x = _tc_call(_node_embed_kernel, (ngrid,),
                 [pl.BlockSpec((NBLK, SH), lambda i: (i, 0)),
                  pl.BlockSpec((NBLK, 9), lambda i: (i, 0))]
                 + [_full(a) for a in ne_args],
                 nspec, jax.ShapeDtypeStruct((n, W), f32),
                 node_s, nv_flat, *ne_args)

    ev_flat = edge_v.swapaxes(1, 2).reshape(e, 3)
    egrid_e = e // EBLK
    egrid = ep // EBLK
    emax = egrid_e - 1
    es2, ev2 = _tc_call(
        _edge_embed_kernel, (egrid_e,),
        [pl.BlockSpec((EBLK, SEW), lambda i: (i, 0)),
         pl.BlockSpec((EBLK, 3), lambda i: (i, 0))]
        + [_full(a) for a in ee_args],
        [pl.BlockSpec((EBLK, SEW), lambda i: (i, 0)),
         pl.BlockSpec((EBLK, 3), lambda i: (i, 0))],
        [jax.ShapeDtypeStruct((e, SEW), f32), jax.ShapeDtypeStruct((e, 3), f32)],
        edge_s, ev_flat, *ee_args)

    ones_ch = jnp.ones((SCH, 128), f32)
    zeros128 = jnp.zeros((rpt, 128), f32)
    cnt_parts = _sc_counts(dst_s, ep, ones_ch, zeros128)
    cnt16 = _tc_call(
        _cnt_kernel, (ngrid,),
        [pl.BlockSpec((2, NBLK, 128), lambda i: (0, i, 0))],
        pl.BlockSpec((NBLK, 16), lambda i: (i, 0)),
        jax.ShapeDtypeStruct((n, 16), f32),
        cnt_parts)
    espec = pl.BlockSpec((EBLK, W), lambda i: (i, 0))

    p256 = pl.BlockSpec((NBLK, 128), lambda i: (i, 0))
    g256 = pl.BlockSpec((EBLK, 128), lambda i: (i, 0))
    for li in range(3):
        lp = P["layer%d" % li]
        pargs = pre_args(lp["conv"])
        a_t, c_t = _tc_call(
            _node_pre_kernel, (ngrid,),
            [nspec] + [_full(w) for w in pargs],
            [p256, p256],
            [jax.ShapeDtypeStruct((n, 128), jnp.int32),
             jax.ShapeDtypeStruct((n, 128), jnp.int32)],
            x, *pargs)
        ga, gc = _sc_gather2(a_t, c_t, src_g, dst_g)
        margs = msg_args(lp["conv"])
        ms, mv = _tc_call(
            _message_kernel, (egrid,),
            [g256, g256,
             pl.BlockSpec((EBLK, SEW), lambda i: (jnp.minimum(i, emax), 0)),
             pl.BlockSpec((EBLK, 3), lambda i: (jnp.minimum(i, emax), 0))]
            + [_full(a) for a in margs],
            [pl.BlockSpec((EBLK, SH), lambda i: (i, 0)),
             pl.BlockSpec((EBLK, 128), lambda i: (i, 0))],
            [jax.ShapeDtypeStruct((ep, SH), f32),
             jax.ShapeDtypeStruct((ep, 128), f32)],
            ga, gc, es2, ev2, *margs)
        parts = _sc_scatter_cols(ms, mv, dst_s, zeros128)
        uargs = upd_args(lp)
        x = _tc_call(
            _node_update_kernel, (ngrid,),
            [nspec,
             pl.BlockSpec((2, NBLK, 128), lambda i: (0, i, 0)),
             pl.BlockSpec((NBLK, 16), lambda i: (i, 0))]
            + [_full(a) for a in uargs],
            nspec, jax.ShapeDtypeStruct((n, W), f32),
            x, parts, cnt16, *uargs)

    av = avaliable_pos.reshape(n, 1)
    pol_pre, val_pre = _tc_call(
        _heads_pre_kernel, (ngrid,),
        [nspec, pl.BlockSpec((NBLK, 1), lambda i: (i, 0))]
        + [_full(a) for a in hp_args],
        [pl.BlockSpec((NBLK, 32), lambda i: (i, 0)),
         pl.BlockSpec((NBLK, 32), lambda i: (i, 0))],
        [jax.ShapeDtypeStruct((n, 32), f32), jax.ShapeDtypeStruct((n, 32), f32)],
        x, av, *hp_args)

    d_in = l * 32
    d_out = 20 * l
    d_h2 = pl.cdiv(d_out, 128) * 128
    rb = d_out // 20
    xTp = jnp.zeros((d_in, 128), f32).at[:, :b].set(
        pol_pre.reshape(b, d_in).T)
    b1m = jnp.broadcast_to(P["pol_fc1"]["b"][:, None], (d_out, 128))
    b2m = jnp.broadcast_to(P["pol_fc2"]["b"][:, None], (d_out, 128))
    w2p = jnp.pad(P["pol_fc2"]["w"], ((0, 0), (0, d_h2 - d_out)))

    h1t = _tc_call(
        functools.partial(_matmul_bias_kernel, act=True), (d_out // rb,),
        [pl.BlockSpec((rb, d_in), lambda j: (j, 0)),
         pl.BlockSpec((d_in, 128), lambda j: (0, 0)),
         pl.BlockSpec((rb, 128), lambda j: (j, 0))],
        pl.BlockSpec((rb, 128), lambda j: (j, 0)),
        jax.ShapeDtypeStruct((d_out, 128), f32),
        P["pol_fc1"]["w"], xTp, b1m)
    h1tp = jnp.pad(h1t, ((0, d_h2 - d_out), (0, 0)))
    pol_t = _tc_call(
        functools.partial(_matmul_bias_kernel, act=False), (d_out // rb,),
        [pl.BlockSpec((rb, d_h2), lambda j: (j, 0)),
         pl.BlockSpec((d_h2, 128), lambda j: (0, 0)),
         pl.BlockSpec((rb, 128), lambda j: (j, 0))],
        pl.BlockSpec((rb, 128), lambda j: (j, 0)),
        jax.ShapeDtypeStruct((d_out, 128), f32),
        w2p, h1tp, b2m)
    pol = pol_t[:, :b].T

    out_size = P["val_fc2"]["b"].shape[0]
    va = (P["val_fc1"]["w"].T, P["val_fc1"]["b"][None, :],
          P["val_fc2"]["w"].T, P["val_fc2"]["b"][None, :])
    val = _tc_call(
        _val_head_kernel, (1,),
        [pl.BlockSpec((b, l, 32), lambda i: (0, 0, 0))] + [_full(a) for a in va],
        pl.BlockSpec((b, out_size), lambda i: (0, 0)),
        jax.ShapeDtypeStruct((b, out_size), f32),
        val_pre.reshape(b, l, 32), *va)

    return pol, val


# speedup vs baseline: 1.3741x; 1.1501x over previous
import functools
import jax
import jax.numpy as jnp
from jax import lax
from jax.experimental import pallas as pl
from jax.experimental.pallas import tpu as pltpu
from jax.experimental.pallas import tpu_sc as plsc

SH = 128
VHC = 16
W = SH + 3 * VHC
SEW = 32
EBLK = 1280
NBLK = 1000
EBLK2 = 2000
GCH = 80
SCH = 128
EPAD = 20480


def _bd3(w):
    return jnp.kron(jnp.eye(3, dtype=w.dtype), w)


def _summ(h):
    return jnp.kron(jnp.ones((3, 1), dtype=jnp.float32), jnp.eye(h, dtype=jnp.float32))


def _ln_s(s, w, b):
    mu = jnp.mean(s, axis=-1, keepdims=True)
    var = jnp.mean((s - mu) * (s - mu), axis=-1, keepdims=True)
    return (s - mu) / jnp.sqrt(var + 1e-5) * w + b


def _dot(x, w):
    return jnp.dot(x, w, preferred_element_type=jnp.float32)


def _vnorm(vh, summ):
    return jnp.sqrt(jnp.clip(_dot(vh * vh, summ), 1e-8, None))


def _vgate(vo, summ):
    sig = jax.nn.sigmoid(_vnorm(vo, summ))
    return vo * jnp.concatenate([sig, sig, sig], axis=1)


def _node_embed_kernel(s_ref, v_ref, lnw, lnb, whb, summ, wss, wsn, wsb, wvb,
                       out_ref):
    s = _ln_s(s_ref[...], lnw[...], lnb[...])
    v = v_ref[...]
    vn = jnp.sqrt(jnp.sum(v * v, axis=-1, keepdims=True) / 3.0 + 1e-8)
    v = v / vn
    vh = _dot(v, whb[...])
    nrm = _vnorm(vh, summ[...])
    so = _dot(s, wss[...]) + _dot(nrm, wsn[...]) + wsb[...]
    vo = _dot(vh, wvb[...])
    out_ref[...] = jnp.concatenate([so, vo], axis=1)


def _edge_embed_kernel(s_ref, v_ref, lnw, lnb, wh00, wss, wsn, wsb, wv00,
                       so_ref, vo_ref):
    s = _ln_s(s_ref[...], lnw[...], lnb[...])
    v = v_ref[...]
    vn = jnp.sqrt(jnp.sum(v * v, axis=-1, keepdims=True) + 1e-8)
    v = v / vn
    vh = v * wh00[0, 0]
    nrm = jnp.sqrt(jnp.clip(jnp.sum(vh * vh, axis=-1, keepdims=True), 1e-8, None))
    so_ref[...] = _dot(s, wss[...]) + _dot(nrm, wsn[...]) + wsb[...]
    vo_ref[...] = vh * wv00[0, 0]


def _pack_bf16_pair(lo, hi):
    lob = lax.bitcast_convert_type(
        lo.astype(jnp.bfloat16).astype(jnp.float32), jnp.int32)
    hib = lax.bitcast_convert_type(
        hi.astype(jnp.bfloat16).astype(jnp.float32), jnp.int32)
    return lax.bitwise_or(lax.shift_right_logical(lob, 16),
                          lax.bitwise_and(hib, jnp.int32(-65536)))


def _unpack_bf16_pair(g32):
    lo = lax.bitcast_convert_type(lax.shift_left(g32, 16), jnp.float32)
    hi = lax.bitcast_convert_type(
        lax.bitwise_and(g32, jnp.int32(-65536)), jnp.float32)
    return lo, hi


def _node_pre_kernel(x_ref, wa, wsrc, wc, wdst, a_ref, c_ref):
    x = x_ref[...]
    s, v = x[:, :SH], x[:, SH:]
    blk = s.shape[0]
    pad = jnp.zeros((blk, SH - 99), jnp.float32)
    a_ref[...] = _pack_bf16_pair(
        _dot(s, wa[...]), jnp.concatenate([_dot(v, wsrc[...]), pad], 1))
    c_ref[...] = _pack_bf16_pair(
        _dot(s, wc[...]), jnp.concatenate([_dot(v, wdst[...]), pad], 1))


def _message_kernel(ga_ref, gc_ref, es_ref, ev_ref,
                    w0ev, summ33, ws0es, ws0n, ws0bias,
                    wv0b, summ16,
                    wh1b, ws1s, ws1n, ws1bias, wv1b,
                    wh2b, ws2s, ws2n, ws2bias, wv2b,
                    ms_ref, mv_ref):
    la, ha = _unpack_bf16_pair(ga_ref[...])
    lc, hc = _unpack_bf16_pair(gc_ref[...])
    es, ev = es_ref[...], ev_ref[...]
    vh0 = (ha + hc)[:, :99] + _dot(ev, w0ev[...])
    nrm0 = _vnorm(vh0, summ33[...])
    s0 = (la + lc + _dot(es, ws0es[...])
          + _dot(nrm0, ws0n[...]) + ws0bias[...])
    s0 = jnp.maximum(s0, 0.0)
    v0 = _vgate(_dot(vh0, wv0b[...]), summ16[...])
    vh1 = _dot(v0, wh1b[...])
    nrm1 = _vnorm(vh1, summ16[...])
    s1 = jnp.maximum(_dot(s0, ws1s[...]) + _dot(nrm1, ws1n[...]) + ws1bias[...], 0.0)
    v1 = _vgate(_dot(vh1, wv1b[...]), summ16[...])
    vh2 = _dot(v1, wh2b[...])
    nrm2 = _vnorm(vh2, summ16[...])
    s2 = _dot(s1, ws2s[...]) + _dot(nrm2, ws2n[...]) + ws2bias[...]
    v2 = _dot(vh2, wv2b[...])
    ms_ref[...] = s2
    mv_ref[...] = jnp.concatenate(
        [v2, jnp.zeros((v2.shape[0], 128 - 3 * VHC), jnp.float32)], axis=1)


def _cnt_kernel(c_ref, out_ref):
    c = c_ref[...]
    out_ref[...] = jnp.maximum(c[0][:, :16], 1.0)


def _node_update_kernel(x_ref, p0_ref, p1_ref, c_ref,
                        ln0w, ln0b,
                        f0wh, summ32, f0wss, f0wsn, f0wsb, f0wv,
                        f1wh, f1wss, f1wsn, f1wsb, f1wv, summ16,
                        ln1w, ln1b,
                        out_ref):
    x = x_ref[...]
    cnt = c_ref[...][:, :1]
    p = p0_ref[...] + p1_ref[...]
    s = x[:, :SH] + p[0] / cnt
    v = x[:, SH:] + p[1][:, :3 * VHC] / cnt
    s = _ln_s(s, ln0w[...], ln0b[...])
    vn = jnp.sqrt(jnp.sum(v * v, axis=-1, keepdims=True) / VHC + 1e-8)
    v = v / vn
    vh = _dot(v, f0wh[...])
    nrm = _vnorm(vh, summ32[...])
    fs = jnp.maximum(_dot(s, f0wss[...]) + _dot(nrm, f0wsn[...]) + f0wsb[...], 0.0)
    fv = _vgate(_dot(vh, f0wv[...]), summ32[...])
    vh1 = _dot(fv, f1wh[...])
    nrm1 = _vnorm(vh1, summ32[...])
    fs1 = _dot(fs, f1wss[...]) + _dot(nrm1, f1wsn[...]) + f1wsb[...]
    fv1 = _dot(vh1, f1wv[...])
    s = s + fs1
    v = v + fv1
    s = _ln_s(s, ln1w[...], ln1b[...])
    vn = jnp.sqrt(jnp.sum(v * v, axis=-1, keepdims=True) / VHC + 1e-8)
    v = v / vn
    out_ref[...] = jnp.concatenate([s, v], axis=1)


def _heads_pre_kernel(x_ref, av_ref,
                      plnw, plnb, pwh, summ16, pwss, pwsn, pwsb,
                      vlnw, vlnb, vwh, vwss, vwsn, vwsb,
                      pol_ref, val_ref):
    x = x_ref[...]
    s, v = x[:, :SH], x[:, SH:]

    def head(lnw, lnb, whb, wss, wsn, wsb):
        s2 = _ln_s(s, lnw, lnb)
        vn = jnp.sqrt(jnp.sum(v * v, axis=-1, keepdims=True) / VHC + 1e-8)
        v2 = v / vn
        vh = _dot(v2, whb)
        nrm = _vnorm(vh, summ16[...])
        return _dot(s2, wss) + _dot(nrm, wsn) + wsb

    pol_ref[...] = head(plnw[...], plnb[...], pwh[...], pwss[...], pwsn[...],
                        pwsb[...]) * av_ref[...]
    val_ref[...] = head(vlnw[...], vlnb[...], vwh[...], vwss[...], vwsn[...],
                        vwsb[...])


def _matmul_bias_kernel(x_ref, w_ref, b_ref, out_ref, *, act):
    h = _dot(x_ref[...], w_ref[...]) + b_ref[...]
    if act:
        h = jnp.maximum(h, 0.0)
    out_ref[...] = h


def _val_head_kernel(vp_ref, w1, b1, w2, b2, out_ref):
    vsum = jnp.sum(vp_ref[...], axis=1)
    h = _dot(vsum, w1[...]) + b1[...]
    h = jnp.where(h > 0, h, 0.01 * h)
    out_ref[...] = _dot(h, w2[...]) + b2[...]


def _sc_gather2(tab_a, tab_c, idx_src, idx_dst):
    n, w = tab_a.shape
    dt = tab_a.dtype
    e = idx_src.shape[0]
    info = plsc.get_sparse_core_info()
    nw = info.num_cores * info.num_subcores
    per_w = e // nw
    n_ch = per_w // GCH
    mesh = plsc.VectorSubcoreMesh(core_axis_name="c", subcore_axis_name="s")

    @functools.partial(
        pl.kernel, mesh=mesh,
        out_type=[jax.ShapeDtypeStruct((e, w), dt),
                  jax.ShapeDtypeStruct((e, w), dt)],
        scratch_types=[pltpu.VMEM((per_w,), jnp.int32),
                       pltpu.VMEM((per_w,), jnp.int32),
                       pltpu.VMEM((GCH, w), dt),
                       pltpu.VMEM((GCH, w), dt),
                       pltpu.VMEM((GCH, w), dt),
                       pltpu.VMEM((GCH, w), dt),
                       pltpu.SemaphoreType.DMA, pltpu.SemaphoreType.DMA,
                       pltpu.SemaphoreType.DMA, pltpu.SemaphoreType.DMA,
                       pltpu.SemaphoreType.DMA, pltpu.SemaphoreType.DMA,
                       pltpu.SemaphoreType.DMA, pltpu.SemaphoreType.DMA],
    )
    def k(ta_h, tc_h, src_h, dst_h, oa_h, oc_h,
          isv, idv, ra0, ra1, rc0, rc1,
          sga0, sga1, sgc0, sgc1, swa0, swa1, swc0, swc1):
        wid = lax.axis_index("c") * info.num_subcores + lax.axis_index("s")
        base = wid * per_w
        ra, rc = (ra0, ra1), (rc0, rc1)
        sga, sgc = (sga0, sga1), (sgc0, sgc1)
        swa, swc = (swa0, swa1), (swc0, swc1)
        pltpu.sync_copy(src_h.at[pl.ds(base, per_w)], isv)
        pltpu.sync_copy(dst_h.at[pl.ds(base, per_w)], idv)

        def start_gather(i, p):
            off = i * GCH
            pltpu.async_copy(ta_h.at[isv.at[pl.ds(off, GCH)]], ra[p], sga[p])
            pltpu.async_copy(tc_h.at[idv.at[pl.ds(off, GCH)]], rc[p], sgc[p])

        def wait_writeouts(p):
            pltpu.make_async_copy(ra[p], oa_h.at[pl.ds(base, GCH)], swa[p]).wait()
            pltpu.make_async_copy(rc[p], oc_h.at[pl.ds(base, GCH)], swc[p]).wait()

        def body(i2, carry):
            i0 = i2 * 2
            for p in (0, 1):
                @pl.when(i2 >= 1)
                def _():
                    wait_writeouts(p)
                start_gather(i0 + p, p)
            for p in (0, 1):
                pltpu.make_async_copy(
                    ta_h.at[isv.at[pl.ds(0, GCH)]], ra[p], sga[p]).wait()
                pltpu.make_async_copy(
                    tc_h.at[idv.at[pl.ds(0, GCH)]], rc[p], sgc[p]).wait()
                off = base + (i0 + p) * GCH
                pltpu.async_copy(ra[p], oa_h.at[pl.ds(off, GCH)], swa[p])
                pltpu.async_copy(rc[p], oc_h.at[pl.ds(off, GCH)], swc[p])
            return carry

        lax.fori_loop(0, n_ch // 2, body, 0)
        for p in (0, 1):
            wait_writeouts(p)

    return k(tab_a, tab_c, idx_src, idx_dst)


def _sc_scatter_cols(ms, mv, idx, zeros_blk):
    e = ms.shape[0]
    info = plsc.get_sparse_core_info()
    ns = info.num_subcores
    per_t = e // ns
    n_ch = per_t // SCH
    rpt = zeros_blk.shape[0]
    npad = rpt * ns
    idx2d = idx.reshape(e // SCH, SCH)
    mesh = plsc.VectorSubcoreMesh(core_axis_name="c", subcore_axis_name="s")

    @functools.partial(
        pl.kernel, mesh=mesh,
        out_type=jax.ShapeDtypeStruct((2, npad, 128), jnp.float32),
        scratch_types=[pltpu.VMEM((n_ch, SCH), jnp.int32),
                       pltpu.VMEM((SCH, 128), jnp.float32),
                       pltpu.VMEM((SCH, 128), jnp.float32),
                       pltpu.VMEM_SHARED((npad, 128), jnp.float32),
                       pltpu.SemaphoreType.DMA, pltpu.SemaphoreType.DMA,
                       pltpu.SemaphoreType.DMA, pltpu.SemaphoreType.DMA],
    )
    def k(ms_h, mv_h, idx_h, zeros_h, out_h, idx_v, r0, r1, acc,
          src0, src1, ssa0, ssa1):
        c = lax.axis_index("c")
        s = lax.axis_index("s")
        rows = (r0, r1)
        src_sem = (src0, src1)
        sa_sem = (ssa0, ssa1)
        pltpu.sync_copy(zeros_h, acc.at[pl.ds(s * rpt, rpt)])
        pltpu.sync_copy(idx_h.at[pl.ds(s * n_ch, n_ch)], idx_v)
        plsc.subcore_barrier()

        def start_rowcopy(j, p):
            b = s * per_t + j * SCH

            @pl.when(c == 0)
            def _():
                pltpu.async_copy(ms_h.at[pl.ds(b, SCH)], rows[p], src_sem[p])

            @pl.when(c == 1)
            def _():
                pltpu.async_copy(mv_h.at[pl.ds(b, SCH)], rows[p], src_sem[p])

        def body(j2, carry):
            j0 = j2 * 2
            for p in (0, 1):
                @pl.when(j2 >= 1)
                def _():
                    pltpu.make_async_copy(
                        rows[p], acc.at[idx_v.at[0]], sa_sem[p]).wait()
                start_rowcopy(j0 + p, p)
            for p in (0, 1):
                pltpu.make_async_copy(
                    ms_h.at[pl.ds(0, SCH)], rows[p], src_sem[p]).wait()
                pltpu.async_copy(rows[p], acc.at[idx_v.at[j0 + p]],
                                 sa_sem[p], add=True)
            return carry

        lax.fori_loop(0, n_ch // 2, body, 0)
        for p in (0, 1):
            pltpu.make_async_copy(rows[p], acc.at[idx_v.at[0]], sa_sem[p]).wait()
        plsc.subcore_barrier()
        pltpu.sync_copy(acc.at[pl.ds(s * rpt, rpt)],
                        out_h.at[c, pl.ds(s * rpt, rpt)])

    return k(ms, mv, idx2d, zeros_blk)


def _sc_counts(idx, e, ones_blk, zeros_blk):
    info = plsc.get_sparse_core_info()
    ns = info.num_subcores
    n_ch = e // (ns * SCH)
    rpt = zeros_blk.shape[0]
    npad = rpt * ns
    mesh = plsc.VectorSubcoreMesh(core_axis_name="c", subcore_axis_name="s")

    @functools.partial(
        pl.kernel, mesh=mesh,
        out_type=jax.ShapeDtypeStruct((2, npad, 128), jnp.float32),
        scratch_types=[pltpu.VMEM((SCH,), jnp.int32),
                       pltpu.VMEM((SCH, 128), jnp.float32),
                       pltpu.VMEM_SHARED((npad, 128), jnp.float32),
                       pltpu.SemaphoreType.DMA],
    )
    def k(idx_h, ones_h, zeros_h, out_h, idx_v, rows_v, acc, sem):
        c = lax.axis_index("c")
        s = lax.axis_index("s")
        pltpu.sync_copy(zeros_h, acc.at[pl.ds(s * rpt, rpt)])
        pltpu.sync_copy(ones_h, rows_v)
        plsc.subcore_barrier()

        def body(i, carry):
            b = (i * ns + s) * SCH
            pltpu.sync_copy(idx_h.at[pl.ds(b, SCH)], idx_v)
            pltpu.sync_copy(rows_v, acc.at[idx_v], add=True)
            return carry

        lax.fori_loop(0, n_ch, body, 0)
        plsc.subcore_barrier()
        pltpu.sync_copy(acc.at[pl.ds(s * rpt, rpt)],
                        out_h.at[c, pl.ds(s * rpt, rpt)])

    return k(idx, ones_blk, zeros_blk)


def _dbg_gather(ta, tc, i_s, i_d):
    return ta[i_s], tc[i_d]


def _dbg_scatter(ms, mv, idx, z):
    npad = z.shape[0] * 16
    s0 = jax.ops.segment_sum(ms, idx, num_segments=npad)
    s1 = jax.ops.segment_sum(mv, idx, num_segments=npad)
    return jnp.stack([s0, s1])


def _dbg_counts(idx, e, ones, z):
    npad = z.shape[0] * 16
    c = jax.ops.segment_sum(jnp.ones((e,), jnp.float32), idx, num_segments=npad)
    c = jnp.broadcast_to(c[:, None], (npad, 128))
    return jnp.stack([c, c])


def _tc_call(body, grid, in_specs, out_specs, out_shape, *args):
    return pl.pallas_call(
        body,
        grid=grid,
        in_specs=in_specs,
        out_specs=out_specs,
        out_shape=out_shape,
    )(*args)


def _full(a):
    return pl.BlockSpec(a.shape, lambda i: tuple(0 for _ in a.shape))


def kernel(node_s, node_v, edge_s, edge_v, avaliable_pos, params, edge_index,
           batch_ids, ptr):
    f32 = jnp.float32
    n = node_s.shape[0]
    e = edge_s.shape[0]
    b = ptr.shape[0] - 1
    l = n // b
    src = edge_index[0].astype(jnp.int32)
    dst = edge_index[1].astype(jnp.int32)
    ep = ((e + EPAD - 1) // EPAD) * EPAD
    rpt = (n // 16 // 8 + 1) * 8
    npad = rpt * 16
    pad = ep - e
    src_g = jnp.concatenate([src, jnp.zeros((pad,), jnp.int32)])
    dst_g = jnp.concatenate([dst, jnp.zeros((pad,), jnp.int32)])
    dst_s = jnp.concatenate([dst, jnp.full((pad,), npad - 1, jnp.int32)])

    P = params
    su16, su32, su33 = _summ(16), _summ(32), _summ(33)

    ng = P["node_gvp"]
    ne_args = (P["node_ln"]["w"][None, :], P["node_ln"]["b"][None, :],
               _bd3(ng["wh"].T), su16,
               ng["ws"]["w"].T[:SH], ng["ws"]["w"].T[SH:],
               ng["ws"]["b"][None, :], _bd3(ng["wv"].T))
    eg = P["edge_gvp"]
    ee_args = (P["edge_ln"]["w"][None, :], P["edge_ln"]["b"][None, :],
               eg["wh"], eg["ws"]["w"].T[:SEW], eg["ws"]["w"].T[SEW:],
               eg["ws"]["b"][None, :], eg["wv"])

    def pre_args(cp):
        m0 = cp["m0"]
        wh0t = m0["wh"].T
        ws0t = m0["ws"]["w"].T
        return (ws0t[0:SH], _bd3(wh0t[0:16]),
                ws0t[SH + SEW:2 * SH + SEW], _bd3(wh0t[17:33]))

    def msg_args(cp):
        m0, m1, m2 = cp["m0"], cp["m1"], cp["m2"]
        wh0t = m0["wh"].T
        ws0t = m0["ws"]["w"].T
        return (_bd3(wh0t[16:17]), su33,
                ws0t[SH:SH + SEW],
                ws0t[2 * SH + SEW:], m0["ws"]["b"][None, :],
                _bd3(m0["wv"].T), su16,
                _bd3(m1["wh"].T), m1["ws"]["w"].T[:SH], m1["ws"]["w"].T[SH:],
                m1["ws"]["b"][None, :], _bd3(m1["wv"].T),
                _bd3(m2["wh"].T), m2["ws"]["w"].T[:SH], m2["ws"]["w"].T[SH:],
                m2["ws"]["b"][None, :], _bd3(m2["wv"].T))

    def upd_args(lp):
        f0, f1 = lp["ff0"], lp["ff1"]
        return (lp["ln0"]["w"][None, :], lp["ln0"]["b"][None, :],
                _bd3(f0["wh"].T), su32,
                f0["ws"]["w"].T[:SH], f0["ws"]["w"].T[SH:],
                f0["ws"]["b"][None, :], _bd3(f0["wv"].T),
                _bd3(f1["wh"].T), f1["ws"]["w"].T[:4 * SH],
                f1["ws"]["w"].T[4 * SH:], f1["ws"]["b"][None, :],
                _bd3(f1["wv"].T), su16,
                lp["ln1"]["w"][None, :], lp["ln1"]["b"][None, :])

    pg, vg = P["pol_gvp"], P["val_gvp"]
    hp_args = (P["pol_ln"]["w"][None, :], P["pol_ln"]["b"][None, :],
               _bd3(pg["wh"].T), su16,
               pg["ws"]["w"].T[:SH], pg["ws"]["w"].T[SH:],
               pg["ws"]["b"][None, :],
               P["val_ln"]["w"][None, :], P["val_ln"]["b"][None, :],
               _bd3(vg["wh"].T),
               vg["ws"]["w"].T[:SH], vg["ws"]["w"].T[SH:],
               vg["ws"]["b"][None, :])

    nv_flat = node_v.swapaxes(1, 2).reshape(n, 9)
    ngrid = n // NBLK
    nspec = pl.BlockSpec((NBLK, W), lambda i: (i, 0))
    x = _tc_call(_node_embed_kernel, (ngrid,),
                 [pl.BlockSpec((NBLK, SH), lambda i: (i, 0)),
                  pl.BlockSpec((NBLK, 9), lambda i: (i, 0))]
                 + [_full(a) for a in ne_args],
                 nspec, jax.ShapeDtypeStruct((n, W), f32),
                 node_s, nv_flat, *ne_args)

    ev_flat = edge_v.swapaxes(1, 2).reshape(e, 3)
    egrid_e = e // EBLK
    egrid = ep // EBLK
    emax = egrid_e - 1
    es2, ev2 = _tc_call(
        _edge_embed_kernel, (egrid_e,),
        [pl.BlockSpec((EBLK, SEW), lambda i: (i, 0)),
         pl.BlockSpec((EBLK, 3), lambda i: (i, 0))]
        + [_full(a) for a in ee_args],
        [pl.BlockSpec((EBLK, SEW), lambda i: (i, 0)),
         pl.BlockSpec((EBLK, 3), lambda i: (i, 0))],
        [jax.ShapeDtypeStruct((e, SEW), f32), jax.ShapeDtypeStruct((e, 3), f32)],
        edge_s, ev_flat, *ee_args)

    ones_ch = jnp.ones((SCH, 128), f32)
    zeros128 = jnp.zeros((rpt, 128), f32)
    cnt_parts = _sc_counts(dst_s, ep, ones_ch, zeros128)
    cnt16 = _tc_call(
        _cnt_kernel, (ngrid,),
        [pl.BlockSpec((2, NBLK, 128), lambda i: (0, i, 0))],
        pl.BlockSpec((NBLK, 16), lambda i: (i, 0)),
        jax.ShapeDtypeStruct((n, 16), f32),
        cnt_parts)
    espec = pl.BlockSpec((EBLK, W), lambda i: (i, 0))

    p256 = pl.BlockSpec((NBLK, 128), lambda i: (i, 0))
    g256 = pl.BlockSpec((EBLK, 128), lambda i: (i, 0))
    eh = ep // 2
    ehb = eh // EBLK
    src_h = (src_g[:eh], src_g[eh:])
    dst_h = (dst_g[:eh], dst_g[eh:])
    dst_sh = (dst_s[:eh], dst_s[eh:])
    for li in range(3):
        lp = P["layer%d" % li]
        pargs = pre_args(lp["conv"])
        a_t, c_t = _tc_call(
            _node_pre_kernel, (ngrid,),
            [nspec] + [_full(w) for w in pargs],
            [p256, p256],
            [jax.ShapeDtypeStruct((n, 128), jnp.int32),
             jax.ShapeDtypeStruct((n, 128), jnp.int32)],
            x, *pargs)
        margs = msg_args(lp["conv"])
        gh = [_sc_gather2(a_t, c_t, src_h[h], dst_h[h]) for h in (0, 1)]
        parts = []
        for h in (0, 1):
            ga, gc = gh[h]
            off = h * ehb
            ms, mv = _tc_call(
                _message_kernel, (ehb,),
                [g256, g256,
                 pl.BlockSpec((EBLK, SEW),
                              lambda i, o=off: (jnp.minimum(i + o, emax), 0)),
                 pl.BlockSpec((EBLK, 3),
                              lambda i, o=off: (jnp.minimum(i + o, emax), 0))]
                + [_full(a) for a in margs],
                [pl.BlockSpec((EBLK, SH), lambda i: (i, 0)),
                 pl.BlockSpec((EBLK, 128), lambda i: (i, 0))],
                [jax.ShapeDtypeStruct((eh, SH), f32),
                 jax.ShapeDtypeStruct((eh, 128), f32)],
                ga, gc, es2, ev2, *margs)
            parts.append(_sc_scatter_cols(ms, mv, dst_sh[h], zeros128))
        uargs = upd_args(lp)
        pspec = pl.BlockSpec((2, NBLK, 128), lambda i: (0, i, 0))
        x = _tc_call(
            _node_update_kernel, (ngrid,),
            [nspec, pspec, pspec,
             pl.BlockSpec((NBLK, 16), lambda i: (i, 0))]
            + [_full(a) for a in uargs],
            nspec, jax.ShapeDtypeStruct((n, W), f32),
            x, parts[0], parts[1], cnt16, *uargs)

    av = avaliable_pos.reshape(n, 1)
    pol_pre, val_pre = _tc_call(
        _heads_pre_kernel, (ngrid,),
        [nspec, pl.BlockSpec((NBLK, 1), lambda i: (i, 0))]
        + [_full(a) for a in hp_args],
        [pl.BlockSpec((NBLK, 32), lambda i: (i, 0)),
         pl.BlockSpec((NBLK, 32), lambda i: (i, 0))],
        [jax.ShapeDtypeStruct((n, 32), f32), jax.ShapeDtypeStruct((n, 32), f32)],
        x, av, *hp_args)

    d_in = l * 32
    d_out = 20 * l
    d_h2 = pl.cdiv(d_out, 128) * 128
    rb = d_out // 20
    xTp = jnp.zeros((d_in, 128), f32).at[:, :b].set(
        pol_pre.reshape(b, d_in).T)
    b1m = jnp.broadcast_to(P["pol_fc1"]["b"][:, None], (d_out, 128))
    b2m = jnp.broadcast_to(P["pol_fc2"]["b"][:, None], (d_out, 128))
    w2p = jnp.pad(P["pol_fc2"]["w"], ((0, 0), (0, d_h2 - d_out)))

    h1t = _tc_call(
        functools.partial(_matmul_bias_kernel, act=True), (d_out // rb,),
        [pl.BlockSpec((rb, d_in), lambda j: (j, 0)),
         pl.BlockSpec((d_in, 128), lambda j: (0, 0)),
         pl.BlockSpec((rb, 128), lambda j: (j, 0))],
        pl.BlockSpec((rb, 128), lambda j: (j, 0)),
        jax.ShapeDtypeStruct((d_out, 128), f32),
        P["pol_fc1"]["w"], xTp, b1m)
    h1tp = jnp.pad(h1t, ((0, d_h2 - d_out), (0, 0)))
    pol_t = _tc_call(
        functools.partial(_matmul_bias_kernel, act=False), (d_out // rb,),
        [pl.BlockSpec((rb, d_h2), lambda j: (j, 0)),
         pl.BlockSpec((d_h2, 128), lambda j: (0, 0)),
         pl.BlockSpec((rb, 128), lambda j: (j, 0))],
        pl.BlockSpec((rb, 128), lambda j: (j, 0)),
        jax.ShapeDtypeStruct((d_out, 128), f32),
        w2p, h1tp, b2m)
    pol = pol_t[:, :b].T

    out_size = P["val_fc2"]["b"].shape[0]
    va = (P["val_fc1"]["w"].T, P["val_fc1"]["b"][None, :],
          P["val_fc2"]["w"].T, P["val_fc2"]["b"][None, :])
    val = _tc_call(
        _val_head_kernel, (1,),
        [pl.BlockSpec((b, l, 32), lambda i: (0, 0, 0))] + [_full(a) for a in va],
        pl.BlockSpec((b, out_size), lambda i: (0, 0)),
        jax.ShapeDtypeStruct((b, out_size), f32),
        val_pre.reshape(b, l, 32), *va)

    return pol, val
